# baseline (speedup 1.0000x reference)
import functools

import jax
import jax.numpy as jnp
from jax import lax
from jax.experimental import pallas as pl
from jax.experimental.pallas import tpu as pltpu
from jax.experimental.pallas import tpu_sc as plsc

BF = jnp.bfloat16
F32 = jnp.float32
EB = 1280
NB = 10000


def _lrelu(v):
    return jnp.where(v >= 0, v, 0.2 * v)


def _sc_gather(table, idx, chunk=80):
    n, d = table.shape
    e = idx.shape[0]
    info = plsc.get_sparse_core_info()
    nw = info.num_cores * info.num_subcores
    per_w = e // nw
    n_chunks = per_w // chunk
    mesh = plsc.VectorSubcoreMesh(core_axis_name="c", subcore_axis_name="s")

    @functools.partial(
        pl.kernel,
        out_type=jax.ShapeDtypeStruct((e, d), table.dtype),
        mesh=mesh,
        scratch_types=[
            pltpu.VMEM((chunk,), jnp.int32),
            pltpu.VMEM((chunk, d), table.dtype),
            pltpu.SemaphoreType.DMA,
        ],
    )
    def k(table_hbm, idx_hbm, out_hbm, idx_v, rows_v, sem):
        wid = lax.axis_index("s") * info.num_cores + lax.axis_index("c")
        base = wid * per_w

        def body(j, carry):
            b = base + j * chunk
            pltpu.sync_copy(idx_hbm.at[pl.ds(b, chunk)], idx_v)
            pltpu.async_copy(table_hbm.at[idx_v], rows_v, sem).wait()
            pltpu.sync_copy(rows_v, out_hbm.at[pl.ds(b, chunk), :])
            return carry

        lax.fori_loop(0, n_chunks, body, 0)

    return k(table, idx)


def _sc_gather_small(table, idx, chunk=80):
    n, d = table.shape
    e = idx.shape[0]
    info = plsc.get_sparse_core_info()
    nw = info.num_cores * info.num_subcores
    per_w = e // nw
    n_chunks = per_w // chunk
    mesh = plsc.VectorSubcoreMesh(core_axis_name="c", subcore_axis_name="s")

    @functools.partial(
        pl.kernel,
        out_type=jax.ShapeDtypeStruct((e * d,), table.dtype),
        mesh=mesh,
        scratch_types=[
            pltpu.VMEM((n * d,), table.dtype),
            pltpu.VMEM((chunk,), jnp.int32),
            pltpu.VMEM((chunk * d,), table.dtype),
        ],
        compiler_params=pltpu.CompilerParams(needs_layout_passes=False),
    )
    def k(table_hbm, idx_hbm, out_hbm, tab_v, idx_v, obuf_v):
        wid = lax.axis_index("s") * info.num_cores + lax.axis_index("c")
        base = wid * per_w
        pltpu.sync_copy(table_hbm, tab_v)

        def body(j, carry):
            b = base + j * chunk
            pltpu.sync_copy(idx_hbm.at[pl.ds(b, chunk)], idx_v)
            for g in range(chunk // 16):
                t = idx_v[pl.ds(g * 16, 16)] * d
                eloc = (lax.iota(jnp.int32, 16) + (g * 16)) * d
                for c in range(4):
                    vals = plsc.load_gather(tab_v, [t + c])
                    plsc.store_scatter(obuf_v, [eloc + c], vals)
            pltpu.sync_copy(obuf_v, out_hbm.at[pl.ds(b * d, chunk * d)])
            return carry

        lax.fori_loop(0, n_chunks, body, 0)

    return k(table.reshape(-1), idx).reshape(e, d)


def _sc_scatter_add(w_t, tgt, n_nodes, chunk=800):
    d, e = w_t.shape
    info = plsc.get_sparse_core_info()
    nc, ns = info.num_cores, info.num_subcores
    cs = d // (nc * ns)
    n_chunks = e // chunk
    zeros = jnp.zeros((cs * n_nodes,), w_t.dtype)
    mesh = plsc.VectorSubcoreMesh(core_axis_name="c", subcore_axis_name="s")

    @functools.partial(
        pl.kernel,
        out_type=jax.ShapeDtypeStruct((d * n_nodes,), w_t.dtype),
        mesh=mesh,
        scratch_types=[
            pltpu.VMEM((2 * chunk,), jnp.int32),
            pltpu.VMEM((2 * cs * chunk,), w_t.dtype),
            pltpu.VMEM((cs * n_nodes,), w_t.dtype),
            pltpu.SemaphoreType.DMA,
            pltpu.SemaphoreType.DMA,
        ],
        compiler_params=pltpu.CompilerParams(needs_layout_passes=False),
    )
    def k(w_hbm, tgt_hbm, zs_hbm, out_hbm, idx_v, rows_v, acc, sem0, sem1):
        cid = lax.axis_index("c")
        sid = lax.axis_index("s")
        col0 = (cid * ns + sid) * cs
        pltpu.sync_copy(zs_hbm, acc)
        sems = [sem0, sem1]

        def fire(j, b):
            eb = j * chunk
            pltpu.async_copy(tgt_hbm.at[pl.ds(eb, chunk)],
                             idx_v.at[pl.ds(b * chunk, chunk)], sems[b])
            for c in range(cs):
                pltpu.async_copy(
                    w_hbm.at[pl.ds((col0 + c) * e + eb, chunk)],
                    rows_v.at[pl.ds((b * cs + c) * chunk, chunk)], sems[b])

        def drain(j, b):
            eb = j * chunk
            pltpu.make_async_copy(
                tgt_hbm.at[pl.ds(eb, chunk)],
                idx_v.at[pl.ds(b * chunk, chunk)], sems[b]).wait()
            for c in range(cs):
                pltpu.make_async_copy(
                    w_hbm.at[pl.ds((col0 + c) * e + eb, chunk)],
                    rows_v.at[pl.ds((b * cs + c) * chunk, chunk)],
                    sems[b]).wait()

        def process(b):
            for g in range(chunk // 16):
                ids = idx_v[pl.ds(b * chunk + g * 16, 16)]
                for c in range(cs):
                    vals = rows_v[pl.ds((b * cs + c) * chunk + g * 16, 16)]
                    plsc.addupdate_scatter(acc, [ids], vals)
                    ids = ids + n_nodes

        fire(0, 0)

        def body(i, carry):
            ja = 2 * i

            @pl.when(ja + 1 < n_chunks)
            def _():
                fire(ja + 1, 1)

            drain(ja, 0)
            process(0)

            @pl.when(ja + 2 < n_chunks)
            def _():
                fire(ja + 2, 0)

            @pl.when(ja + 1 < n_chunks)
            def _():
                drain(ja + 1, 1)
                process(1)

            return carry

        lax.fori_loop(0, (n_chunks + 1) // 2, body, 0)
        for c in range(cs):
            pltpu.sync_copy(
                acc.at[pl.ds(c * n_nodes, n_nodes)],
                out_hbm.at[pl.ds((col0 + c) * n_nodes, n_nodes)])

    return k(w_t.reshape(-1), tgt, zeros).reshape(d, n_nodes)


def _dot(a, b):
    return jax.lax.dot(a.astype(BF), b.astype(BF), preferred_element_type=F32)


def _dot_t(a_t, b):
    return jax.lax.dot_general(
        a_t.astype(BF), b.astype(BF),
        dimension_numbers=(((0,), (0,)), ((), ())),
        preferred_element_type=F32)


def _row(i):
    return (i, 0)


def _row3(i):
    return (i, 0, 0)


def _fix(i):
    return (0, 0)


def _specs(block_rows, row_shapes, fixed_shapes):
    sp = [pl.BlockSpec((block_rows, s), _row) for s in row_shapes]
    sp += [pl.BlockSpec(s, _fix) for s in fixed_shapes]
    return sp


def _pass_a_body(nb, ea, tht, w1a, w1b, b1, w2, awm, bmax_ref, bsum_ref,
                 lg_ref, lbf_ref, c2_ref):
    i = pl.program_id(0)

    @pl.when(i == 0)
    def _():
        c2_ref[...] = jnp.transpose(_dot(w2[...], awm[...]))

    pre = _dot(nb[...], w1a[...]) + _dot(ea[...], w1b[...]) + b1[...]
    L = _lrelu(pre)
    lbf_ref[...] = L.astype(BF)
    c2t = c2_ref[...]
    lg = jnp.concatenate(
        [jnp.sum(L * c2t[hh:hh + 1, :], axis=1, keepdims=True)
         for hh in range(4)], axis=1) + tht[...][:, :4]
    lg_ref[...] = lg
    mx = jnp.max(lg, axis=0, keepdims=True)
    sm = jnp.sum(jnp.exp(lg - mx), axis=0, keepdims=True)
    bmax_ref[...] = mx[None]
    bsum_ref[...] = sm[None]


def _pass_a(nb, ea, tht, w1a, w1b, b1, w2, awm):
    e = nb.shape[0]
    g = e // EB
    hh = w2.shape[1]
    return pl.pallas_call(
        _pass_a_body,
        grid=(g,),
        in_specs=_specs(EB, [nb.shape[1], ea.shape[1], tht.shape[1]],
                        [w1a.shape, w1b.shape, b1.shape, w2.shape, awm.shape]),
        out_specs=[pl.BlockSpec((1, 1, 4), _row3),
                   pl.BlockSpec((1, 1, 4), _row3),
                   pl.BlockSpec((EB, 4), _row),
                   pl.BlockSpec((EB, hh), _row)],
        out_shape=[jax.ShapeDtypeStruct((g, 1, 4), F32),
                   jax.ShapeDtypeStruct((g, 1, 4), F32),
                   jax.ShapeDtypeStruct((e, 4), F32),
                   jax.ShapeDtypeStruct((e, hh), BF)],
        scratch_shapes=[pltpu.VMEM((4, w2.shape[1]), F32)],
    )(nb, ea, tht, w1a, w1b, b1, w2, awm)


def _pass_c_body(lbf, lg, bmax, bsum, w2, b2, w_ref):
    m = jax.lax.dot(lbf[...], w2[...].astype(BF),
                    preferred_element_type=F32) + b2[...]
    bm = bmax[...][:, 0, :]
    bs = bsum[...][:, 0, :]
    gmax = jnp.max(bm, axis=0, keepdims=True)
    z = jnp.sum(bs * jnp.exp(bm - gmax), axis=0, keepdims=True)
    a = jnp.exp(lg[...] - gmax) / z
    h = w2.shape[1] // 4
    acc = a[:, 0:1] * m[:, :h]
    for hh in range(1, 4):
        acc = acc + a[:, hh:hh + 1] * m[:, hh * h:(hh + 1) * h]
    w_ref[...] = jnp.transpose(0.25 * acc)


def _pass_c(lbf, lg, bmax, bsum, w2, b2):
    e = lbf.shape[0]
    g = e // EB
    h = w2.shape[1] // 4
    return pl.pallas_call(
        _pass_c_body,
        grid=(g,),
        in_specs=([pl.BlockSpec((EB, lbf.shape[1]), _row),
                   pl.BlockSpec((EB, 4), _row),
                   pl.BlockSpec(bmax.shape, lambda i: (0, 0, 0)),
                   pl.BlockSpec(bsum.shape, lambda i: (0, 0, 0))] +
                  [pl.BlockSpec(x.shape, _fix) for x in (w2, b2)]),
        out_specs=pl.BlockSpec((h, EB), lambda i: (0, i)),
        out_shape=jax.ShapeDtypeStruct((h, e), F32),
    )(lbf, lg, bmax, bsum, w2, b2)


def _node_body(hb, aggb_t, u1, u2, ub, lng, lnb, awh, hn_ref, hbf_ref,
               thp_ref):
    u = _dot(hb[...], u1[...]) + _dot_t(aggb_t[...], u2[...]) + ub[...]
    mu = jnp.mean(u, axis=-1, keepdims=True)
    d = u - mu
    var = jnp.mean(d * d, axis=-1, keepdims=True)
    un = d / jnp.sqrt(var + 1e-5) * lng[...] + lnb[...]
    hn = hb[...] + _lrelu(un)
    hn_ref[...] = hn
    hbf_ref[...] = hn.astype(BF)
    th = _dot(hn, awh[...])
    thp_ref[...] = jnp.concatenate(
        [th, jnp.zeros((th.shape[0], 4), F32)], axis=-1)


def _node_pass(h, agg_t, u1, u2, ub, lng, lnb, awh):
    n, hd = h.shape
    g = n // NB
    return pl.pallas_call(
        _node_body,
        grid=(g,),
        in_specs=([pl.BlockSpec((NB, hd), _row),
                   pl.BlockSpec((hd, NB), lambda i: (0, i))] +
                  [pl.BlockSpec(x.shape, _fix)
                   for x in (u1, u2, ub, lng, lnb, awh)]),
        out_specs=[pl.BlockSpec((NB, hd), _row), pl.BlockSpec((NB, hd), _row),
                   pl.BlockSpec((NB, 8), _row)],
        out_shape=[jax.ShapeDtypeStruct((n, hd), F32),
                   jax.ShapeDtypeStruct((n, hd), BF),
                   jax.ShapeDtypeStruct((n, 8), F32)],
    )(h, agg_t, u1, u2, ub, lng, lnb, awh)


def _init_body(xb, inw, inb, awh, h_ref, hbf_ref, thp_ref):
    h = _dot(xb[...], inw[...]) + inb[...]
    h_ref[...] = h
    hbf_ref[...] = h.astype(BF)
    th = _dot(h, awh[...])
    thp_ref[...] = jnp.concatenate(
        [th, jnp.zeros((th.shape[0], 4), F32)], axis=-1)


def _init_pass(x, inw, inb, awh):
    n, din = x.shape
    hd = inw.shape[1]
    g = n // NB
    return pl.pallas_call(
        _init_body,
        grid=(g,),
        in_specs=_specs(NB, [din], [inw.shape, inb.shape, awh.shape]),
        out_specs=[pl.BlockSpec((NB, hd), _row), pl.BlockSpec((NB, hd), _row),
                   pl.BlockSpec((NB, 8), _row)],
        out_shape=[jax.ShapeDtypeStruct((n, hd), F32),
                   jax.ShapeDtypeStruct((n, hd), BF),
                   jax.ShapeDtypeStruct((n, 8), F32)],
    )(x, inw, inb, awh)


def _final_body(hb, tw1, tb1, tw2, tb2, gw1, gb1, gw2, gb2, t_ref, g_ref,
                acc_ref):
    i = pl.program_id(0)
    n_total = pl.num_programs(0) * hb.shape[0]
    t1 = jnp.maximum(_dot(hb[...], tw1[...]) + tb1[...], 0.0)
    t_ref[...] = _dot(t1, tw2[...]) + tb2[...]

    @pl.when(i == 0)
    def _():
        acc_ref[...] = jnp.zeros_like(acc_ref)

    acc_ref[...] += jnp.sum(hb[...], axis=0, keepdims=True)

    @pl.when(i == pl.num_programs(0) - 1)
    def _():
        ge = acc_ref[...] / n_total
        g1 = jnp.maximum(_dot(ge, gw1[...]) + gb1[...], 0.0)
        g_ref[...] = _dot(g1, gw2[...]) + gb2[...]


def _final_pass(h, tw1, tb1, tw2, tb2, gw1, gb1, gw2, gb2):
    n, hd = h.shape
    g = n // NB
    return pl.pallas_call(
        _final_body,
        grid=(g,),
        in_specs=_specs(NB, [hd],
                        [tw1.shape, tb1.shape, tw2.shape, tb2.shape,
                         gw1.shape, gb1.shape, gw2.shape, gb2.shape]),
        out_specs=[pl.BlockSpec((NB, 1), _row),
                   pl.BlockSpec((1, 4), lambda i: (0, 0))],
        out_shape=[jax.ShapeDtypeStruct((n, 1), F32),
                   jax.ShapeDtypeStruct((1, 4), F32)],
        scratch_shapes=[pltpu.VMEM((1, hd), F32)],
    )(h, tw1, tb1, tw2, tb2, gw1, gb1, gw2, gb2)


def kernel(x, edge_index, edge_attr, params):
    src, tgt = edge_index[0], edge_index[1]
    hd = params['inW'].shape[1]
    n = x.shape[0]
    layers = params['layers']

    h, hbf, thp = _init_pass(x, params['inW'], params['inb'].reshape(1, -1),
                             layers[0]['aW'][layers[0]['mW2'].shape[1]:])
    for li, lp in enumerate(layers):
        hh = lp['mW2'].shape[1]
        w1a = lp['mW1'][:hd]
        w1b = lp['mW1'][hd:]
        b1 = lp['mb1'].reshape(1, -1)
        b2 = lp['mb2'].reshape(1, -1)
        awm = lp['aW'][:hh]
        ab = lp['ab'].reshape(1, -1)

        hb32 = lax.bitcast_convert_type(
            hbf.reshape(n, hd // 2, 2), jnp.int32)
        nb32 = _sc_gather(hb32, src)
        nb = lax.bitcast_convert_type(nb32, jnp.bfloat16).reshape(-1, hd)
        tht = _sc_gather_small(thp, tgt)
        bmax, bsum, lg, lbf = _pass_a(nb, edge_attr, tht, w1a, w1b, b1,
                                      lp['mW2'], awm)
        w = _pass_c(lbf, lg, bmax, bsum, lp['mW2'], b2)
        agg = _sc_scatter_add(w, tgt, n)
        nxt = layers[li + 1] if li + 1 < len(layers) else None
        awh_next = (nxt['aW'][nxt['mW2'].shape[1]:] if nxt is not None
                    else jnp.zeros((hd, 4), F32))
        h, hbf, thp = _node_pass(h, agg, lp['uW'][:hd], lp['uW'][hd:],
                            lp['ub'].reshape(1, -1),
                            lp['ln_g'].reshape(1, -1),
                            lp['ln_b'].reshape(1, -1), awh_next)

    t, g = _final_pass(h, params['tW1'], params['tb1'].reshape(1, -1),
                       params['tW2'], params['tb2'].reshape(1, -1),
                       params['gW1'], params['gb1'].reshape(1, -1),
                       params['gW2'], params['gb2'].reshape(1, -1))
    return t.reshape(-1), h, g.reshape(-1)


# speedup vs baseline: 1.0439x; 1.0439x over previous
import functools

import jax
import jax.numpy as jnp
from jax import lax
from jax.experimental import pallas as pl
from jax.experimental.pallas import tpu as pltpu
from jax.experimental.pallas import tpu_sc as plsc

BF = jnp.bfloat16
F32 = jnp.float32
EB = 1280
NB = 10000


def _lrelu(v):
    return jnp.where(v >= 0, v, 0.2 * v)


def _sc_gather(table, idx, chunk=80):
    n, d = table.shape
    e = idx.shape[0]
    info = plsc.get_sparse_core_info()
    nw = info.num_cores * info.num_subcores
    per_w = e // nw
    n_chunks = per_w // chunk
    mesh = plsc.VectorSubcoreMesh(core_axis_name="c", subcore_axis_name="s")

    @functools.partial(
        pl.kernel,
        out_type=jax.ShapeDtypeStruct((e, d), table.dtype),
        mesh=mesh,
        scratch_types=[
            pltpu.VMEM((chunk,), jnp.int32),
            pltpu.VMEM((chunk, d), table.dtype),
            pltpu.SemaphoreType.DMA,
        ],
    )
    def k(table_hbm, idx_hbm, out_hbm, idx_v, rows_v, sem):
        wid = lax.axis_index("s") * info.num_cores + lax.axis_index("c")
        base = wid * per_w

        def body(j, carry):
            b = base + j * chunk
            pltpu.sync_copy(idx_hbm.at[pl.ds(b, chunk)], idx_v)
            pltpu.async_copy(table_hbm.at[idx_v], rows_v, sem).wait()
            pltpu.sync_copy(rows_v, out_hbm.at[pl.ds(b, chunk), :])
            return carry

        lax.fori_loop(0, n_chunks, body, 0)

    return k(table, idx)


def _sc_gather_small(table, idx, chunk=80):
    n, d = table.shape
    e = idx.shape[0]
    info = plsc.get_sparse_core_info()
    nw = info.num_cores * info.num_subcores
    per_w = e // nw
    n_chunks = per_w // chunk
    mesh = plsc.VectorSubcoreMesh(core_axis_name="c", subcore_axis_name="s")

    @functools.partial(
        pl.kernel,
        out_type=jax.ShapeDtypeStruct((e * d,), table.dtype),
        mesh=mesh,
        scratch_types=[
            pltpu.VMEM((n * d,), table.dtype),
            pltpu.VMEM((chunk,), jnp.int32),
            pltpu.VMEM((chunk * d,), table.dtype),
        ],
        compiler_params=pltpu.CompilerParams(needs_layout_passes=False),
    )
    def k(table_hbm, idx_hbm, out_hbm, tab_v, idx_v, obuf_v):
        wid = lax.axis_index("s") * info.num_cores + lax.axis_index("c")
        base = wid * per_w
        pltpu.sync_copy(table_hbm, tab_v)

        def body(j, carry):
            b = base + j * chunk
            pltpu.sync_copy(idx_hbm.at[pl.ds(b, chunk)], idx_v)
            for g in range(chunk // 16):
                t = idx_v[pl.ds(g * 16, 16)] * d
                eloc = (lax.iota(jnp.int32, 16) + (g * 16)) * d
                for c in range(4):
                    vals = plsc.load_gather(tab_v, [t + c])
                    plsc.store_scatter(obuf_v, [eloc + c], vals)
            pltpu.sync_copy(obuf_v, out_hbm.at[pl.ds(b * d, chunk * d)])
            return carry

        lax.fori_loop(0, n_chunks, body, 0)

    return k(table.reshape(-1), idx).reshape(e, d)


def _sc_scatter_add(w_t, tgt, n_nodes, chunk=800):
    d, e = w_t.shape
    info = plsc.get_sparse_core_info()
    nc, ns = info.num_cores, info.num_subcores
    cs = d // (nc * ns)
    n_chunks = e // chunk
    zeros = jnp.zeros((cs * n_nodes,), w_t.dtype)
    mesh = plsc.VectorSubcoreMesh(core_axis_name="c", subcore_axis_name="s")

    @functools.partial(
        pl.kernel,
        out_type=jax.ShapeDtypeStruct((d * n_nodes,), w_t.dtype),
        mesh=mesh,
        scratch_types=[
            pltpu.VMEM((2 * chunk,), jnp.int32),
            pltpu.VMEM((2 * cs * chunk,), w_t.dtype),
            pltpu.VMEM((cs * n_nodes,), w_t.dtype),
            pltpu.SemaphoreType.DMA,
            pltpu.SemaphoreType.DMA,
        ],
        compiler_params=pltpu.CompilerParams(needs_layout_passes=False),
    )
    def k(w_hbm, tgt_hbm, zs_hbm, out_hbm, idx_v, rows_v, acc, sem0, sem1):
        cid = lax.axis_index("c")
        sid = lax.axis_index("s")
        col0 = (cid * ns + sid) * cs
        pltpu.sync_copy(zs_hbm, acc)
        sems = [sem0, sem1]

        def fire(j, b):
            eb = j * chunk
            pltpu.async_copy(tgt_hbm.at[pl.ds(eb, chunk)],
                             idx_v.at[pl.ds(b * chunk, chunk)], sems[b])
            for c in range(cs):
                pltpu.async_copy(
                    w_hbm.at[pl.ds((col0 + c) * e + eb, chunk)],
                    rows_v.at[pl.ds((b * cs + c) * chunk, chunk)], sems[b])

        def drain(j, b):
            eb = j * chunk
            pltpu.make_async_copy(
                tgt_hbm.at[pl.ds(eb, chunk)],
                idx_v.at[pl.ds(b * chunk, chunk)], sems[b]).wait()
            for c in range(cs):
                pltpu.make_async_copy(
                    w_hbm.at[pl.ds((col0 + c) * e + eb, chunk)],
                    rows_v.at[pl.ds((b * cs + c) * chunk, chunk)],
                    sems[b]).wait()

        def process(b):
            for g in range(chunk // 16):
                ids = idx_v[pl.ds(b * chunk + g * 16, 16)]
                for c in range(cs):
                    vals = rows_v[pl.ds((b * cs + c) * chunk + g * 16, 16)]
                    plsc.addupdate_scatter(acc, [ids], vals)
                    ids = ids + n_nodes

        fire(0, 0)

        def body(i, carry):
            ja = 2 * i

            @pl.when(ja + 1 < n_chunks)
            def _():
                fire(ja + 1, 1)

            drain(ja, 0)
            process(0)

            @pl.when(ja + 2 < n_chunks)
            def _():
                fire(ja + 2, 0)

            @pl.when(ja + 1 < n_chunks)
            def _():
                drain(ja + 1, 1)
                process(1)

            return carry

        lax.fori_loop(0, (n_chunks + 1) // 2, body, 0)
        for c in range(cs):
            pltpu.sync_copy(
                acc.at[pl.ds(c * n_nodes, n_nodes)],
                out_hbm.at[pl.ds((col0 + c) * n_nodes, n_nodes)])

    return k(w_t.reshape(-1), tgt, zeros).reshape(d, n_nodes)


def _dot(a, b):
    return jax.lax.dot(a.astype(BF), b.astype(BF), preferred_element_type=F32)


def _dot_t(a_t, b):
    return jax.lax.dot_general(
        a_t.astype(BF), b.astype(BF),
        dimension_numbers=(((0,), (0,)), ((), ())),
        preferred_element_type=F32)


def _row(i):
    return (i, 0)


def _row3(i):
    return (i, 0, 0)


def _fix(i):
    return (0, 0)


def _specs(block_rows, row_shapes, fixed_shapes):
    sp = [pl.BlockSpec((block_rows, s), _row) for s in row_shapes]
    sp += [pl.BlockSpec(s, _fix) for s in fixed_shapes]
    return sp


def _pass_a_body(nb, ea, tht, w1a, w1b, b1, w2, awm, bmax_ref, bsum_ref,
                 lg_ref, lbf_ref, c2_ref):
    i = pl.program_id(0)

    @pl.when(i == 0)
    def _():
        c2_ref[...] = _dot(w2[...], awm[...])

    pre = _dot(nb[...], w1a[...]) + _dot(ea[...], w1b[...]) + b1[...]
    L = _lrelu(pre)
    lbf_ref[...] = L.astype(BF)
    lg = _dot(L, c2_ref[...]) + tht[...][:, :4]
    lg_ref[...] = lg
    mx = jnp.max(lg, axis=0, keepdims=True)
    sm = jnp.sum(jnp.exp(lg - mx), axis=0, keepdims=True)
    bmax_ref[...] = mx[None]
    bsum_ref[...] = sm[None]


def _pass_a(nb, ea, tht, w1a, w1b, b1, w2, awm):
    e = nb.shape[0]
    g = e // EB
    hh = w2.shape[1]
    return pl.pallas_call(
        _pass_a_body,
        grid=(g,),
        in_specs=_specs(EB, [nb.shape[1], ea.shape[1], tht.shape[1]],
                        [w1a.shape, w1b.shape, b1.shape, w2.shape, awm.shape]),
        out_specs=[pl.BlockSpec((1, 1, 4), _row3),
                   pl.BlockSpec((1, 1, 4), _row3),
                   pl.BlockSpec((EB, 4), _row),
                   pl.BlockSpec((EB, hh), _row)],
        out_shape=[jax.ShapeDtypeStruct((g, 1, 4), F32),
                   jax.ShapeDtypeStruct((g, 1, 4), F32),
                   jax.ShapeDtypeStruct((e, 4), F32),
                   jax.ShapeDtypeStruct((e, hh), BF)],
        scratch_shapes=[pltpu.VMEM((w2.shape[1], 4), F32)],
    )(nb, ea, tht, w1a, w1b, b1, w2, awm)


def _pass_c_body(lbf, lg, bmax, bsum, w2, b2, w_ref):
    m = jax.lax.dot(lbf[...], w2[...].astype(BF),
                    preferred_element_type=F32) + b2[...]
    bm = bmax[...][:, 0, :]
    bs = bsum[...][:, 0, :]
    gmax = jnp.max(bm, axis=0, keepdims=True)
    z = jnp.sum(bs * jnp.exp(bm - gmax), axis=0, keepdims=True)
    a = jnp.exp(lg[...] - gmax) / z
    h = w2.shape[1] // 4
    acc = a[:, 0:1] * m[:, :h]
    for hh in range(1, 4):
        acc = acc + a[:, hh:hh + 1] * m[:, hh * h:(hh + 1) * h]
    w_ref[...] = jnp.transpose(0.25 * acc)


def _pass_c(lbf, lg, bmax, bsum, w2, b2):
    e = lbf.shape[0]
    g = e // EB
    h = w2.shape[1] // 4
    return pl.pallas_call(
        _pass_c_body,
        grid=(g,),
        in_specs=([pl.BlockSpec((EB, lbf.shape[1]), _row),
                   pl.BlockSpec((EB, 4), _row),
                   pl.BlockSpec(bmax.shape, lambda i: (0, 0, 0)),
                   pl.BlockSpec(bsum.shape, lambda i: (0, 0, 0))] +
                  [pl.BlockSpec(x.shape, _fix) for x in (w2, b2)]),
        out_specs=pl.BlockSpec((h, EB), lambda i: (0, i)),
        out_shape=jax.ShapeDtypeStruct((h, e), F32),
    )(lbf, lg, bmax, bsum, w2, b2)


def _node_body(hb, aggb_t, u1, u2, ub, lng, lnb, awh, hn_ref, hbf_ref,
               thp_ref):
    u = _dot(hb[...], u1[...]) + _dot_t(aggb_t[...], u2[...]) + ub[...]
    mu = jnp.mean(u, axis=-1, keepdims=True)
    d = u - mu
    var = jnp.mean(d * d, axis=-1, keepdims=True)
    un = d / jnp.sqrt(var + 1e-5) * lng[...] + lnb[...]
    hn = hb[...] + _lrelu(un)
    hn_ref[...] = hn
    hbf_ref[...] = hn.astype(BF)
    th = _dot(hn, awh[...])
    thp_ref[...] = jnp.concatenate(
        [th, jnp.zeros((th.shape[0], 4), F32)], axis=-1)


def _node_pass(h, agg_t, u1, u2, ub, lng, lnb, awh):
    n, hd = h.shape
    g = n // NB
    return pl.pallas_call(
        _node_body,
        grid=(g,),
        in_specs=([pl.BlockSpec((NB, hd), _row),
                   pl.BlockSpec((hd, NB), lambda i: (0, i))] +
                  [pl.BlockSpec(x.shape, _fix)
                   for x in (u1, u2, ub, lng, lnb, awh)]),
        out_specs=[pl.BlockSpec((NB, hd), _row), pl.BlockSpec((NB, hd), _row),
                   pl.BlockSpec((NB, 8), _row)],
        out_shape=[jax.ShapeDtypeStruct((n, hd), F32),
                   jax.ShapeDtypeStruct((n, hd), BF),
                   jax.ShapeDtypeStruct((n, 8), F32)],
    )(h, agg_t, u1, u2, ub, lng, lnb, awh)


def _init_body(xb, inw, inb, awh, h_ref, hbf_ref, thp_ref):
    h = _dot(xb[...], inw[...]) + inb[...]
    h_ref[...] = h
    hbf_ref[...] = h.astype(BF)
    th = _dot(h, awh[...])
    thp_ref[...] = jnp.concatenate(
        [th, jnp.zeros((th.shape[0], 4), F32)], axis=-1)


def _init_pass(x, inw, inb, awh):
    n, din = x.shape
    hd = inw.shape[1]
    g = n // NB
    return pl.pallas_call(
        _init_body,
        grid=(g,),
        in_specs=_specs(NB, [din], [inw.shape, inb.shape, awh.shape]),
        out_specs=[pl.BlockSpec((NB, hd), _row), pl.BlockSpec((NB, hd), _row),
                   pl.BlockSpec((NB, 8), _row)],
        out_shape=[jax.ShapeDtypeStruct((n, hd), F32),
                   jax.ShapeDtypeStruct((n, hd), BF),
                   jax.ShapeDtypeStruct((n, 8), F32)],
    )(x, inw, inb, awh)


def _final_body(hb, tw1, tb1, tw2, tb2, gw1, gb1, gw2, gb2, t_ref, g_ref,
                acc_ref):
    i = pl.program_id(0)
    n_total = pl.num_programs(0) * hb.shape[0]
    t1 = jnp.maximum(_dot(hb[...], tw1[...]) + tb1[...], 0.0)
    t_ref[...] = _dot(t1, tw2[...]) + tb2[...]

    @pl.when(i == 0)
    def _():
        acc_ref[...] = jnp.zeros_like(acc_ref)

    acc_ref[...] += jnp.sum(hb[...], axis=0, keepdims=True)

    @pl.when(i == pl.num_programs(0) - 1)
    def _():
        ge = acc_ref[...] / n_total
        g1 = jnp.maximum(_dot(ge, gw1[...]) + gb1[...], 0.0)
        g_ref[...] = _dot(g1, gw2[...]) + gb2[...]


def _final_pass(h, tw1, tb1, tw2, tb2, gw1, gb1, gw2, gb2):
    n, hd = h.shape
    g = n // NB
    return pl.pallas_call(
        _final_body,
        grid=(g,),
        in_specs=_specs(NB, [hd],
                        [tw1.shape, tb1.shape, tw2.shape, tb2.shape,
                         gw1.shape, gb1.shape, gw2.shape, gb2.shape]),
        out_specs=[pl.BlockSpec((NB, 1), _row),
                   pl.BlockSpec((1, 4), lambda i: (0, 0))],
        out_shape=[jax.ShapeDtypeStruct((n, 1), F32),
                   jax.ShapeDtypeStruct((1, 4), F32)],
        scratch_shapes=[pltpu.VMEM((1, hd), F32)],
    )(h, tw1, tb1, tw2, tb2, gw1, gb1, gw2, gb2)


def kernel(x, edge_index, edge_attr, params):
    src, tgt = edge_index[0], edge_index[1]
    hd = params['inW'].shape[1]
    n = x.shape[0]
    layers = params['layers']

    h, hbf, thp = _init_pass(x, params['inW'], params['inb'].reshape(1, -1),
                             layers[0]['aW'][layers[0]['mW2'].shape[1]:])
    for li, lp in enumerate(layers):
        hh = lp['mW2'].shape[1]
        w1a = lp['mW1'][:hd]
        w1b = lp['mW1'][hd:]
        b1 = lp['mb1'].reshape(1, -1)
        b2 = lp['mb2'].reshape(1, -1)
        awm = lp['aW'][:hh]
        ab = lp['ab'].reshape(1, -1)

        hb32 = lax.bitcast_convert_type(
            hbf.reshape(n, hd // 2, 2), jnp.int32)
        nb32 = _sc_gather(hb32, src)
        nb = lax.bitcast_convert_type(nb32, jnp.bfloat16).reshape(-1, hd)
        tht = _sc_gather_small(thp, tgt)
        bmax, bsum, lg, lbf = _pass_a(nb, edge_attr, tht, w1a, w1b, b1,
                                      lp['mW2'], awm)
        w = _pass_c(lbf, lg, bmax, bsum, lp['mW2'], b2)
        agg = _sc_scatter_add(w, tgt, n)
        nxt = layers[li + 1] if li + 1 < len(layers) else None
        awh_next = (nxt['aW'][nxt['mW2'].shape[1]:] if nxt is not None
                    else jnp.zeros((hd, 4), F32))
        h, hbf, thp = _node_pass(h, agg, lp['uW'][:hd], lp['uW'][hd:],
                            lp['ub'].reshape(1, -1),
                            lp['ln_g'].reshape(1, -1),
                            lp['ln_b'].reshape(1, -1), awh_next)

    t, g = _final_pass(h, params['tW1'], params['tb1'].reshape(1, -1),
                       params['tW2'], params['tb2'].reshape(1, -1),
                       params['gW1'], params['gb1'].reshape(1, -1),
                       params['gW2'], params['gb2'].reshape(1, -1))
    return t.reshape(-1), h, g.reshape(-1)


# speedup vs baseline: 1.4381x; 1.3776x over previous
import functools

import jax
import jax.numpy as jnp
from jax import lax
from jax.experimental import pallas as pl
from jax.experimental.pallas import tpu as pltpu
from jax.experimental.pallas import tpu_sc as plsc

BF = jnp.bfloat16
F32 = jnp.float32
EB = 1280
NB = 10000


def _lrelu(v):
    return jnp.where(v >= 0, v, 0.2 * v)


def _sc_gather(table, idx, chunk=80):
    n, d = table.shape
    e = idx.shape[0]
    info = plsc.get_sparse_core_info()
    nw = info.num_cores * info.num_subcores
    per_w = e // nw
    n_chunks = per_w // chunk
    mesh = plsc.VectorSubcoreMesh(core_axis_name="c", subcore_axis_name="s")

    @functools.partial(
        pl.kernel,
        out_type=jax.ShapeDtypeStruct((e, d), table.dtype),
        mesh=mesh,
        scratch_types=[
            pltpu.VMEM((chunk,), jnp.int32),
            pltpu.VMEM((chunk, d), table.dtype),
            pltpu.SemaphoreType.DMA,
        ],
    )
    def k(table_hbm, idx_hbm, out_hbm, idx_v, rows_v, sem):
        wid = lax.axis_index("s") * info.num_cores + lax.axis_index("c")
        base = wid * per_w

        def body(j, carry):
            b = base + j * chunk
            pltpu.sync_copy(idx_hbm.at[pl.ds(b, chunk)], idx_v)
            pltpu.async_copy(table_hbm.at[idx_v], rows_v, sem).wait()
            pltpu.sync_copy(rows_v, out_hbm.at[pl.ds(b, chunk), :])
            return carry

        lax.fori_loop(0, n_chunks, body, 0)

    return k(table, idx)


def _sc_gather_small(table, idx, chunk=80):
    n, d = table.shape
    e = idx.shape[0]
    info = plsc.get_sparse_core_info()
    nw = info.num_cores * info.num_subcores
    per_w = e // nw
    n_chunks = per_w // chunk
    mesh = plsc.VectorSubcoreMesh(core_axis_name="c", subcore_axis_name="s")

    @functools.partial(
        pl.kernel,
        out_type=jax.ShapeDtypeStruct((e * d,), table.dtype),
        mesh=mesh,
        scratch_types=[
            pltpu.VMEM((n * d,), table.dtype),
            pltpu.VMEM((chunk,), jnp.int32),
            pltpu.VMEM((chunk * d,), table.dtype),
        ],
        compiler_params=pltpu.CompilerParams(needs_layout_passes=False),
    )
    def k(table_hbm, idx_hbm, out_hbm, tab_v, idx_v, obuf_v):
        wid = lax.axis_index("s") * info.num_cores + lax.axis_index("c")
        base = wid * per_w
        pltpu.sync_copy(table_hbm, tab_v)

        def body(j, carry):
            b = base + j * chunk
            pltpu.sync_copy(idx_hbm.at[pl.ds(b, chunk)], idx_v)
            for g in range(chunk // 16):
                t = idx_v[pl.ds(g * 16, 16)] * d
                eloc = (lax.iota(jnp.int32, 16) + (g * 16)) * d
                for c in range(4):
                    vals = plsc.load_gather(tab_v, [t + c])
                    plsc.store_scatter(obuf_v, [eloc + c], vals)
            pltpu.sync_copy(obuf_v, out_hbm.at[pl.ds(b * d, chunk * d)])
            return carry

        lax.fori_loop(0, n_chunks, body, 0)

    return k(table.reshape(-1), idx).reshape(e, d)


def _sc_scatter_add(w_t, tgt, n_nodes, chunk=800):
    d, e = w_t.shape
    info = plsc.get_sparse_core_info()
    nc, ns = info.num_cores, info.num_subcores
    cs = d // (nc * ns)
    n_chunks = e // chunk
    zeros = jnp.zeros((cs * n_nodes,), w_t.dtype)
    mesh = plsc.VectorSubcoreMesh(core_axis_name="c", subcore_axis_name="s")

    @functools.partial(
        pl.kernel,
        out_type=jax.ShapeDtypeStruct((d * n_nodes,), w_t.dtype),
        mesh=mesh,
        scratch_types=[
            pltpu.VMEM((2 * chunk,), jnp.int32),
            pltpu.VMEM((2 * cs * chunk,), w_t.dtype),
            pltpu.VMEM((cs * n_nodes,), w_t.dtype),
            pltpu.SemaphoreType.DMA,
            pltpu.SemaphoreType.DMA,
        ],
        compiler_params=pltpu.CompilerParams(needs_layout_passes=False),
    )
    def k(w_hbm, tgt_hbm, zs_hbm, out_hbm, idx_v, rows_v, acc, sem0, sem1):
        cid = lax.axis_index("c")
        sid = lax.axis_index("s")
        col0 = (cid * ns + sid) * cs
        pltpu.sync_copy(zs_hbm, acc)
        sems = [sem0, sem1]

        def fire(j, b):
            eb = j * chunk
            pltpu.async_copy(tgt_hbm.at[pl.ds(eb, chunk)],
                             idx_v.at[pl.ds(b * chunk, chunk)], sems[b])
            for c in range(cs):
                pltpu.async_copy(
                    w_hbm.at[pl.ds((col0 + c) * e + eb, chunk)],
                    rows_v.at[pl.ds((b * cs + c) * chunk, chunk)], sems[b])

        def drain(j, b):
            eb = j * chunk
            pltpu.make_async_copy(
                tgt_hbm.at[pl.ds(eb, chunk)],
                idx_v.at[pl.ds(b * chunk, chunk)], sems[b]).wait()
            for c in range(cs):
                pltpu.make_async_copy(
                    w_hbm.at[pl.ds((col0 + c) * e + eb, chunk)],
                    rows_v.at[pl.ds((b * cs + c) * chunk, chunk)],
                    sems[b]).wait()

        def process(b):
            for g in range(chunk // 16):
                ids = idx_v[pl.ds(b * chunk + g * 16, 16)]
                for c in range(cs):
                    vals = rows_v[pl.ds((b * cs + c) * chunk + g * 16, 16)]
                    plsc.addupdate_scatter(acc, [ids], vals)
                    ids = ids + n_nodes

        fire(0, 0)

        def body(i, carry):
            ja = 2 * i

            @pl.when(ja + 1 < n_chunks)
            def _():
                fire(ja + 1, 1)

            drain(ja, 0)
            process(0)

            @pl.when(ja + 2 < n_chunks)
            def _():
                fire(ja + 2, 0)

            @pl.when(ja + 1 < n_chunks)
            def _():
                drain(ja + 1, 1)
                process(1)

            return carry

        lax.fori_loop(0, (n_chunks + 1) // 2, body, 0)
        for c in range(cs):
            pltpu.sync_copy(
                acc.at[pl.ds(c * n_nodes, n_nodes)],
                out_hbm.at[pl.ds((col0 + c) * n_nodes, n_nodes)])

    return k(w_t.reshape(-1), tgt, zeros).reshape(d, n_nodes)


def _dot(a, b):
    return jax.lax.dot(a.astype(BF), b.astype(BF), preferred_element_type=F32)


def _dot_t(a_t, b):
    return jax.lax.dot_general(
        a_t.astype(BF), b.astype(BF),
        dimension_numbers=(((0,), (0,)), ((), ())),
        preferred_element_type=F32)


def _row(i):
    return (i, 0)


def _row3(i):
    return (i, 0, 0)


def _fix(i):
    return (0, 0)


def _specs(block_rows, row_shapes, fixed_shapes):
    sp = [pl.BlockSpec((block_rows, s), _row) for s in row_shapes]
    sp += [pl.BlockSpec(s, _fix) for s in fixed_shapes]
    return sp


def _pass_a_body(nb, ea, tht, w1a, w1b, b1, w2, awm, bmax_ref, bsum_ref,
                 lg_ref, lbf_ref, c2_ref):
    i = pl.program_id(0)

    @pl.when(i == 0)
    def _():
        c2_ref[...] = _dot(w2[...], awm[...])

    pre = _dot(nb[...], w1a[...]) + _dot(ea[...], w1b[...]) + b1[...]
    L = _lrelu(pre)
    lbf_ref[...] = L.astype(BF)
    lg = _dot(L, c2_ref[...]) + tht[...][:, :4]
    lg_ref[...] = lg
    mx = jnp.max(lg, axis=0, keepdims=True)
    sm = jnp.sum(jnp.exp(lg - mx), axis=0, keepdims=True)
    bmax_ref[...] = mx[None]
    bsum_ref[...] = sm[None]


def _pass_a(nb, ea, tht, w1a, w1b, b1, w2, awm):
    e = nb.shape[0]
    g = e // EB
    hh = w2.shape[1]
    return pl.pallas_call(
        _pass_a_body,
        grid=(g,),
        in_specs=_specs(EB, [nb.shape[1], ea.shape[1], tht.shape[1]],
                        [w1a.shape, w1b.shape, b1.shape, w2.shape, awm.shape]),
        out_specs=[pl.BlockSpec((1, 1, 4), _row3),
                   pl.BlockSpec((1, 1, 4), _row3),
                   pl.BlockSpec((EB, 4), _row),
                   pl.BlockSpec((EB, hh), _row)],
        out_shape=[jax.ShapeDtypeStruct((g, 1, 4), F32),
                   jax.ShapeDtypeStruct((g, 1, 4), F32),
                   jax.ShapeDtypeStruct((e, 4), F32),
                   jax.ShapeDtypeStruct((e, hh), BF)],
        scratch_shapes=[pltpu.VMEM((w2.shape[1], 4), F32)],
    )(nb, ea, tht, w1a, w1b, b1, w2, awm)


def _pass_c_body(lbf, lg, bmax, bsum, w2, b2, w_ref):
    m = jax.lax.dot(lbf[...], w2[...].astype(BF),
                    preferred_element_type=F32) + b2[...]
    bm = bmax[...][:, 0, :]
    bs = bsum[...][:, 0, :]
    gmax = jnp.max(bm, axis=0, keepdims=True)
    z = jnp.sum(bs * jnp.exp(bm - gmax), axis=0, keepdims=True)
    a = jnp.exp(lg[...] - gmax) / z
    h = w2.shape[1] // 4
    acc = a[:, 0:1] * m[:, :h]
    for hh in range(1, 4):
        acc = acc + a[:, hh:hh + 1] * m[:, hh * h:(hh + 1) * h]
    w_ref[...] = jnp.transpose(0.25 * acc)


def _pass_c(lbf, lg, bmax, bsum, w2, b2):
    e = lbf.shape[0]
    g = e // EB
    h = w2.shape[1] // 4
    return pl.pallas_call(
        _pass_c_body,
        grid=(g,),
        in_specs=([pl.BlockSpec((EB, lbf.shape[1]), _row),
                   pl.BlockSpec((EB, 4), _row),
                   pl.BlockSpec(bmax.shape, lambda i: (0, 0, 0)),
                   pl.BlockSpec(bsum.shape, lambda i: (0, 0, 0))] +
                  [pl.BlockSpec(x.shape, _fix) for x in (w2, b2)]),
        out_specs=pl.BlockSpec((h, EB), lambda i: (0, i)),
        out_shape=jax.ShapeDtypeStruct((h, e), F32),
    )(lbf, lg, bmax, bsum, w2, b2)


def _node_body(hb, aggb_t, u1, u2, ub, lng, lnb, awh, hn_ref, hbf_ref,
               thp_ref):
    u = _dot(hb[...], u1[...]) + _dot_t(aggb_t[...], u2[...]) + ub[...]
    mu = jnp.mean(u, axis=-1, keepdims=True)
    d = u - mu
    var = jnp.mean(d * d, axis=-1, keepdims=True)
    un = d / jnp.sqrt(var + 1e-5) * lng[...] + lnb[...]
    hn = hb[...] + _lrelu(un)
    hn_ref[...] = hn
    hbf_ref[...] = hn.astype(BF)
    th = _dot(hn, awh[...])
    thp_ref[...] = jnp.concatenate(
        [th, jnp.zeros((th.shape[0], 4), F32)], axis=-1)


def _node_pass(h, agg_t, u1, u2, ub, lng, lnb, awh):
    n, hd = h.shape
    g = n // NB
    return pl.pallas_call(
        _node_body,
        grid=(g,),
        in_specs=([pl.BlockSpec((NB, hd), _row),
                   pl.BlockSpec((hd, NB), lambda i: (0, i))] +
                  [pl.BlockSpec(x.shape, _fix)
                   for x in (u1, u2, ub, lng, lnb, awh)]),
        out_specs=[pl.BlockSpec((NB, hd), _row), pl.BlockSpec((NB, hd), _row),
                   pl.BlockSpec((NB, 8), _row)],
        out_shape=[jax.ShapeDtypeStruct((n, hd), F32),
                   jax.ShapeDtypeStruct((n, hd), BF),
                   jax.ShapeDtypeStruct((n, 8), F32)],
    )(h, agg_t, u1, u2, ub, lng, lnb, awh)


def _init_body(xb, inw, inb, awh, h_ref, hbf_ref, thp_ref):
    h = _dot(xb[...], inw[...]) + inb[...]
    h_ref[...] = h
    hbf_ref[...] = h.astype(BF)
    th = _dot(h, awh[...])
    thp_ref[...] = jnp.concatenate(
        [th, jnp.zeros((th.shape[0], 4), F32)], axis=-1)


def _init_pass(x, inw, inb, awh):
    n, din = x.shape
    hd = inw.shape[1]
    g = n // NB
    return pl.pallas_call(
        _init_body,
        grid=(g,),
        in_specs=_specs(NB, [din], [inw.shape, inb.shape, awh.shape]),
        out_specs=[pl.BlockSpec((NB, hd), _row), pl.BlockSpec((NB, hd), _row),
                   pl.BlockSpec((NB, 8), _row)],
        out_shape=[jax.ShapeDtypeStruct((n, hd), F32),
                   jax.ShapeDtypeStruct((n, hd), BF),
                   jax.ShapeDtypeStruct((n, 8), F32)],
    )(x, inw, inb, awh)


def _final_body(hb, tw1, tb1, tw2, tb2, gw1, gb1, gw2, gb2, t_ref, g_ref,
                acc_ref):
    i = pl.program_id(0)
    n_total = pl.num_programs(0) * hb.shape[0]
    t1 = jnp.maximum(_dot(hb[...], tw1[...]) + tb1[...], 0.0)
    t_ref[...] = _dot(t1, tw2[...]) + tb2[...]

    @pl.when(i == 0)
    def _():
        acc_ref[...] = jnp.zeros_like(acc_ref)

    acc_ref[...] += jnp.sum(hb[...], axis=0, keepdims=True)

    @pl.when(i == pl.num_programs(0) - 1)
    def _():
        ge = acc_ref[...] / n_total
        g1 = jnp.maximum(_dot(ge, gw1[...]) + gb1[...], 0.0)
        g_ref[...] = _dot(g1, gw2[...]) + gb2[...]


def _final_pass(h, tw1, tb1, tw2, tb2, gw1, gb1, gw2, gb2):
    n, hd = h.shape
    g = n // NB
    return pl.pallas_call(
        _final_body,
        grid=(g,),
        in_specs=_specs(NB, [hd],
                        [tw1.shape, tb1.shape, tw2.shape, tb2.shape,
                         gw1.shape, gb1.shape, gw2.shape, gb2.shape]),
        out_specs=[pl.BlockSpec((NB, 1), _row),
                   pl.BlockSpec((1, 4), lambda i: (0, 0))],
        out_shape=[jax.ShapeDtypeStruct((n, 1), F32),
                   jax.ShapeDtypeStruct((1, 4), F32)],
        scratch_shapes=[pltpu.VMEM((1, hd), F32)],
    )(h, tw1, tb1, tw2, tb2, gw1, gb1, gw2, gb2)


def kernel(x, edge_index, edge_attr, params):
    src, tgt = edge_index[0], edge_index[1]
    hd = params['inW'].shape[1]
    n = x.shape[0]
    layers = params['layers']

    h, hbf, thp = _init_pass(x, params['inW'], params['inb'].reshape(1, -1),
                             layers[0]['aW'][layers[0]['mW2'].shape[1]:])
    for li, lp in enumerate(layers):
        hh = lp['mW2'].shape[1]
        w1a = lp['mW1'][:hd]
        w1b = lp['mW1'][hd:]
        b1 = lp['mb1'].reshape(1, -1)
        b2 = lp['mb2'].reshape(1, -1)
        awm = lp['aW'][:hh]
        ab = lp['ab'].reshape(1, -1)

        nb = _sc_gather(h, src)
        tht = _sc_gather_small(thp, tgt)
        bmax, bsum, lg, lbf = _pass_a(nb, edge_attr, tht, w1a, w1b, b1,
                                      lp['mW2'], awm)
        w = _pass_c(lbf, lg, bmax, bsum, lp['mW2'], b2)
        agg = _sc_scatter_add(w, tgt, n)
        nxt = layers[li + 1] if li + 1 < len(layers) else None
        awh_next = (nxt['aW'][nxt['mW2'].shape[1]:] if nxt is not None
                    else jnp.zeros((hd, 4), F32))
        h, hbf, thp = _node_pass(h, agg, lp['uW'][:hd], lp['uW'][hd:],
                            lp['ub'].reshape(1, -1),
                            lp['ln_g'].reshape(1, -1),
                            lp['ln_b'].reshape(1, -1), awh_next)

    t, g = _final_pass(h, params['tW1'], params['tb1'].reshape(1, -1),
                       params['tW2'], params['tb2'].reshape(1, -1),
                       params['gW1'], params['gb1'].reshape(1, -1),
                       params['gW2'], params['gb2'].reshape(1, -1))
    return t.reshape(-1), h, g.reshape(-1)


# speedup vs baseline: 1.5976x; 1.1109x over previous
import functools

import jax
import jax.numpy as jnp
from jax import lax
from jax.experimental import pallas as pl
from jax.experimental.pallas import tpu as pltpu
from jax.experimental.pallas import tpu_sc as plsc

BF = jnp.bfloat16
F32 = jnp.float32
EB = 1280
NB = 10000


def _lrelu(v):
    return jnp.where(v >= 0, v, 0.2 * v)


def _sc_gather(table, idx, chunk=80):
    n, d = table.shape
    e = idx.shape[0]
    info = plsc.get_sparse_core_info()
    nw = info.num_cores * info.num_subcores
    per_w = e // nw
    n_chunks = per_w // chunk
    mesh = plsc.VectorSubcoreMesh(core_axis_name="c", subcore_axis_name="s")

    @functools.partial(
        pl.kernel,
        out_type=jax.ShapeDtypeStruct((e, d), table.dtype),
        mesh=mesh,
        scratch_types=[
            pltpu.VMEM((chunk,), jnp.int32),
            pltpu.VMEM((chunk, d), table.dtype),
            pltpu.SemaphoreType.DMA,
        ],
    )
    def k(table_hbm, idx_hbm, out_hbm, idx_v, rows_v, sem):
        wid = lax.axis_index("s") * info.num_cores + lax.axis_index("c")
        base = wid * per_w

        def body(j, carry):
            b = base + j * chunk
            pltpu.sync_copy(idx_hbm.at[pl.ds(b, chunk)], idx_v)
            pltpu.async_copy(table_hbm.at[idx_v], rows_v, sem).wait()
            pltpu.sync_copy(rows_v, out_hbm.at[pl.ds(b, chunk), :])
            return carry

        lax.fori_loop(0, n_chunks, body, 0)

    return k(table, idx)


def _sc_gather_small(table, idx, chunk=80):
    n, d = table.shape
    e = idx.shape[0]
    info = plsc.get_sparse_core_info()
    nw = info.num_cores * info.num_subcores
    per_w = e // nw
    n_chunks = per_w // chunk
    mesh = plsc.VectorSubcoreMesh(core_axis_name="c", subcore_axis_name="s")

    @functools.partial(
        pl.kernel,
        out_type=jax.ShapeDtypeStruct((e * d,), table.dtype),
        mesh=mesh,
        scratch_types=[
            pltpu.VMEM((n * d,), table.dtype),
            pltpu.VMEM((chunk,), jnp.int32),
            pltpu.VMEM((chunk * d,), table.dtype),
        ],
        compiler_params=pltpu.CompilerParams(needs_layout_passes=False),
    )
    def k(table_hbm, idx_hbm, out_hbm, tab_v, idx_v, obuf_v):
        wid = lax.axis_index("s") * info.num_cores + lax.axis_index("c")
        base = wid * per_w
        pltpu.sync_copy(table_hbm, tab_v)

        def body(j, carry):
            b = base + j * chunk
            pltpu.sync_copy(idx_hbm.at[pl.ds(b, chunk)], idx_v)
            for g in range(chunk // 16):
                t = idx_v[pl.ds(g * 16, 16)] * d
                eloc = (lax.iota(jnp.int32, 16) + (g * 16)) * d
                for c in range(4):
                    vals = plsc.load_gather(tab_v, [t + c])
                    plsc.store_scatter(obuf_v, [eloc + c], vals)
            pltpu.sync_copy(obuf_v, out_hbm.at[pl.ds(b * d, chunk * d)])
            return carry

        lax.fori_loop(0, n_chunks, body, 0)

    return k(table.reshape(-1), idx).reshape(e, d)


def _sc_scatter_add(w_t, tgt, n_nodes, chunk=800):
    d, e = w_t.shape
    info = plsc.get_sparse_core_info()
    nc, ns = info.num_cores, info.num_subcores
    cs = d // (nc * ns)
    n_chunks = e // chunk
    zeros = jnp.zeros((cs * n_nodes,), w_t.dtype)
    mesh = plsc.VectorSubcoreMesh(core_axis_name="c", subcore_axis_name="s")

    @functools.partial(
        pl.kernel,
        out_type=jax.ShapeDtypeStruct((d * n_nodes,), w_t.dtype),
        mesh=mesh,
        scratch_types=[
            pltpu.VMEM((2 * chunk,), jnp.int32),
            pltpu.VMEM((2 * cs * chunk,), w_t.dtype),
            pltpu.VMEM((cs * n_nodes,), w_t.dtype),
            pltpu.SemaphoreType.DMA,
            pltpu.SemaphoreType.DMA,
        ],
        compiler_params=pltpu.CompilerParams(needs_layout_passes=False),
    )
    def k(w_hbm, tgt_hbm, zs_hbm, out_hbm, idx_v, rows_v, acc, sem0, sem1):
        cid = lax.axis_index("c")
        sid = lax.axis_index("s")
        col0 = (cid * ns + sid) * cs
        pltpu.sync_copy(zs_hbm, acc)
        sems = [sem0, sem1]

        def fire(j, b):
            eb = j * chunk
            pltpu.async_copy(tgt_hbm.at[pl.ds(eb, chunk)],
                             idx_v.at[pl.ds(b * chunk, chunk)], sems[b])
            for c in range(cs):
                pltpu.async_copy(
                    w_hbm.at[pl.ds((col0 + c) * e + eb, chunk)],
                    rows_v.at[pl.ds((b * cs + c) * chunk, chunk)], sems[b])

        def drain(j, b):
            eb = j * chunk
            pltpu.make_async_copy(
                tgt_hbm.at[pl.ds(eb, chunk)],
                idx_v.at[pl.ds(b * chunk, chunk)], sems[b]).wait()
            for c in range(cs):
                pltpu.make_async_copy(
                    w_hbm.at[pl.ds((col0 + c) * e + eb, chunk)],
                    rows_v.at[pl.ds((b * cs + c) * chunk, chunk)],
                    sems[b]).wait()

        def process(b):
            for g in range(chunk // 16):
                ids = idx_v[pl.ds(b * chunk + g * 16, 16)]
                for c in range(cs):
                    vals = rows_v[pl.ds((b * cs + c) * chunk + g * 16, 16)]
                    plsc.addupdate_scatter(acc, [ids], vals)
                    ids = ids + n_nodes

        fire(0, 0)

        def body(i, carry):
            ja = 2 * i

            @pl.when(ja + 1 < n_chunks)
            def _():
                fire(ja + 1, 1)

            drain(ja, 0)
            process(0)

            @pl.when(ja + 2 < n_chunks)
            def _():
                fire(ja + 2, 0)

            @pl.when(ja + 1 < n_chunks)
            def _():
                drain(ja + 1, 1)
                process(1)

            return carry

        lax.fori_loop(0, (n_chunks + 1) // 2, body, 0)
        for c in range(cs):
            pltpu.sync_copy(
                acc.at[pl.ds(c * n_nodes, n_nodes)],
                out_hbm.at[pl.ds((col0 + c) * n_nodes, n_nodes)])

    return k(w_t.reshape(-1), tgt, zeros).reshape(d, n_nodes)


def _dot(a, b):
    return jax.lax.dot(a.astype(BF), b.astype(BF), preferred_element_type=F32)


def _dot_t(a_t, b):
    return jax.lax.dot_general(
        a_t.astype(BF), b.astype(BF),
        dimension_numbers=(((0,), (0,)), ((), ())),
        preferred_element_type=F32)


def _row(i):
    return (i, 0)


def _row3(i):
    return (i, 0, 0)


def _fix(i):
    return (0, 0)


def _specs(block_rows, row_shapes, fixed_shapes):
    sp = [pl.BlockSpec((block_rows, s), _row) for s in row_shapes]
    sp += [pl.BlockSpec(s, _fix) for s in fixed_shapes]
    return sp


def _pass_a_body(nb, ea, tht, w1a, w1b, b1, w2, awm, bmax_ref, bsum_ref,
                 lg_ref, lbf_ref, c2_ref):
    i = pl.program_id(0)

    @pl.when(i == 0)
    def _():
        c2_ref[...] = _dot(w2[...], awm[...])

    pre = _dot(nb[...], w1a[...]) + _dot(ea[...], w1b[...]) + b1[...]
    L = _lrelu(pre)
    lbf_ref[...] = L.astype(BF)
    lg = _dot(L, c2_ref[...]) + tht[...][:, :4]
    lg_ref[...] = lg
    mx = jnp.max(lg, axis=0, keepdims=True)
    sm = jnp.sum(jnp.exp(lg - mx), axis=0, keepdims=True)
    bmax_ref[...] = mx[None]
    bsum_ref[...] = sm[None]


def _pass_a(nb, ea, tht, w1a, w1b, b1, w2, awm):
    e = nb.shape[0]
    g = e // EB
    hh = w2.shape[1]
    return pl.pallas_call(
        _pass_a_body,
        grid=(g,),
        in_specs=_specs(EB, [nb.shape[1], ea.shape[1], tht.shape[1]],
                        [w1a.shape, w1b.shape, b1.shape, w2.shape, awm.shape]),
        out_specs=[pl.BlockSpec((1, 1, 4), _row3),
                   pl.BlockSpec((1, 1, 4), _row3),
                   pl.BlockSpec((EB, 4), _row),
                   pl.BlockSpec((EB, hh), _row)],
        out_shape=[jax.ShapeDtypeStruct((g, 1, 4), F32),
                   jax.ShapeDtypeStruct((g, 1, 4), F32),
                   jax.ShapeDtypeStruct((e, 4), F32),
                   jax.ShapeDtypeStruct((e, hh), BF)],
        scratch_shapes=[pltpu.VMEM((w2.shape[1], 4), F32)],
    )(nb, ea, tht, w1a, w1b, b1, w2, awm)


def _pass_c_body(lbf, lg, bmax, bsum, w2, b2, w_ref):
    m = jax.lax.dot(lbf[...], w2[...].astype(BF),
                    preferred_element_type=F32) + b2[...]
    bm = bmax[...][:, 0, :]
    bs = bsum[...][:, 0, :]
    gmax = jnp.max(bm, axis=0, keepdims=True)
    z = jnp.sum(bs * jnp.exp(bm - gmax), axis=0, keepdims=True)
    a = jnp.exp(lg[...] - gmax) / z
    h = w2.shape[1] // 4
    acc = a[:, 0:1] * m[:, :h]
    for hh in range(1, 4):
        acc = acc + a[:, hh:hh + 1] * m[:, hh * h:(hh + 1) * h]
    w_ref[...] = jnp.transpose(0.25 * acc)


def _pass_c(lbf, lg, bmax, bsum, w2, b2, off, e_half):
    g = e_half // EB
    h = w2.shape[1] // 4
    row_o = lambda i: (i + off, 0)
    return pl.pallas_call(
        _pass_c_body,
        grid=(g,),
        in_specs=([pl.BlockSpec((EB, lbf.shape[1]), row_o),
                   pl.BlockSpec((EB, 4), row_o),
                   pl.BlockSpec(bmax.shape, lambda i: (0, 0, 0)),
                   pl.BlockSpec(bsum.shape, lambda i: (0, 0, 0))] +
                  [pl.BlockSpec(x.shape, _fix) for x in (w2, b2)]),
        out_specs=pl.BlockSpec((h, EB), lambda i: (0, i)),
        out_shape=jax.ShapeDtypeStruct((h, e_half), F32),
    )(lbf, lg, bmax, bsum, w2, b2)


def _node_body(hb, agg1_t, agg2_t, u1, u2, ub, lng, lnb, awh, hn_ref,
               hbf_ref, thp_ref):
    u = (_dot(hb[...], u1[...]) +
         _dot_t(agg1_t[...] + agg2_t[...], u2[...]) + ub[...])
    mu = jnp.mean(u, axis=-1, keepdims=True)
    d = u - mu
    var = jnp.mean(d * d, axis=-1, keepdims=True)
    un = d / jnp.sqrt(var + 1e-5) * lng[...] + lnb[...]
    hn = hb[...] + _lrelu(un)
    hn_ref[...] = hn
    hbf_ref[...] = hn.astype(BF)
    th = _dot(hn, awh[...])
    thp_ref[...] = jnp.concatenate(
        [th, jnp.zeros((th.shape[0], 4), F32)], axis=-1)


def _node_pass(h, agg1_t, agg2_t, u1, u2, ub, lng, lnb, awh):
    n, hd = h.shape
    g = n // NB
    return pl.pallas_call(
        _node_body,
        grid=(g,),
        in_specs=([pl.BlockSpec((NB, hd), _row),
                   pl.BlockSpec((hd, NB), lambda i: (0, i)),
                   pl.BlockSpec((hd, NB), lambda i: (0, i))] +
                  [pl.BlockSpec(x.shape, _fix)
                   for x in (u1, u2, ub, lng, lnb, awh)]),
        out_specs=[pl.BlockSpec((NB, hd), _row), pl.BlockSpec((NB, hd), _row),
                   pl.BlockSpec((NB, 8), _row)],
        out_shape=[jax.ShapeDtypeStruct((n, hd), F32),
                   jax.ShapeDtypeStruct((n, hd), BF),
                   jax.ShapeDtypeStruct((n, 8), F32)],
        compiler_params=pltpu.CompilerParams(
            vmem_limit_bytes=100 * 1024 * 1024),
    )(h, agg1_t, agg2_t, u1, u2, ub, lng, lnb, awh)


def _init_body(xb, inw, inb, awh, h_ref, hbf_ref, thp_ref):
    h = _dot(xb[...], inw[...]) + inb[...]
    h_ref[...] = h
    hbf_ref[...] = h.astype(BF)
    th = _dot(h, awh[...])
    thp_ref[...] = jnp.concatenate(
        [th, jnp.zeros((th.shape[0], 4), F32)], axis=-1)


def _init_pass(x, inw, inb, awh):
    n, din = x.shape
    hd = inw.shape[1]
    g = n // NB
    return pl.pallas_call(
        _init_body,
        grid=(g,),
        in_specs=_specs(NB, [din], [inw.shape, inb.shape, awh.shape]),
        out_specs=[pl.BlockSpec((NB, hd), _row), pl.BlockSpec((NB, hd), _row),
                   pl.BlockSpec((NB, 8), _row)],
        out_shape=[jax.ShapeDtypeStruct((n, hd), F32),
                   jax.ShapeDtypeStruct((n, hd), BF),
                   jax.ShapeDtypeStruct((n, 8), F32)],
    )(x, inw, inb, awh)


def _final_body(hb, tw1, tb1, tw2, tb2, gw1, gb1, gw2, gb2, t_ref, g_ref,
                acc_ref):
    i = pl.program_id(0)
    n_total = pl.num_programs(0) * hb.shape[0]
    t1 = jnp.maximum(_dot(hb[...], tw1[...]) + tb1[...], 0.0)
    t_ref[...] = _dot(t1, tw2[...]) + tb2[...]

    @pl.when(i == 0)
    def _():
        acc_ref[...] = jnp.zeros_like(acc_ref)

    acc_ref[...] += jnp.sum(hb[...], axis=0, keepdims=True)

    @pl.when(i == pl.num_programs(0) - 1)
    def _():
        ge = acc_ref[...] / n_total
        g1 = jnp.maximum(_dot(ge, gw1[...]) + gb1[...], 0.0)
        g_ref[...] = _dot(g1, gw2[...]) + gb2[...]


def _final_pass(h, tw1, tb1, tw2, tb2, gw1, gb1, gw2, gb2):
    n, hd = h.shape
    g = n // NB
    return pl.pallas_call(
        _final_body,
        grid=(g,),
        in_specs=_specs(NB, [hd],
                        [tw1.shape, tb1.shape, tw2.shape, tb2.shape,
                         gw1.shape, gb1.shape, gw2.shape, gb2.shape]),
        out_specs=[pl.BlockSpec((NB, 1), _row),
                   pl.BlockSpec((1, 4), lambda i: (0, 0))],
        out_shape=[jax.ShapeDtypeStruct((n, 1), F32),
                   jax.ShapeDtypeStruct((1, 4), F32)],
        scratch_shapes=[pltpu.VMEM((1, hd), F32)],
    )(h, tw1, tb1, tw2, tb2, gw1, gb1, gw2, gb2)


def kernel(x, edge_index, edge_attr, params):
    src, tgt = edge_index[0], edge_index[1]
    hd = params['inW'].shape[1]
    n = x.shape[0]
    layers = params['layers']

    h, hbf, thp = _init_pass(x, params['inW'], params['inb'].reshape(1, -1),
                             layers[0]['aW'][layers[0]['mW2'].shape[1]:])
    for li, lp in enumerate(layers):
        hh = lp['mW2'].shape[1]
        w1a = lp['mW1'][:hd]
        w1b = lp['mW1'][hd:]
        b1 = lp['mb1'].reshape(1, -1)
        b2 = lp['mb2'].reshape(1, -1)
        awm = lp['aW'][:hh]
        ab = lp['ab'].reshape(1, -1)

        nb = _sc_gather(h, src)
        tht = _sc_gather_small(thp, tgt)
        bmax, bsum, lg, lbf = _pass_a(nb, edge_attr, tht, w1a, w1b, b1,
                                      lp['mW2'], awm)
        e2 = edge_attr.shape[0] // 2
        g2 = e2 // EB
        w1t = _pass_c(lbf, lg, bmax, bsum, lp['mW2'], b2, 0, e2)
        agg1 = _sc_scatter_add(w1t, tgt[:e2], n)
        w2t = _pass_c(lbf, lg, bmax, bsum, lp['mW2'], b2, g2, e2)
        agg2 = _sc_scatter_add(w2t, tgt[e2:], n)
        nxt = layers[li + 1] if li + 1 < len(layers) else None
        awh_next = (nxt['aW'][nxt['mW2'].shape[1]:] if nxt is not None
                    else jnp.zeros((hd, 4), F32))
        h, hbf, thp = _node_pass(h, agg1, agg2, lp['uW'][:hd], lp['uW'][hd:],
                            lp['ub'].reshape(1, -1),
                            lp['ln_g'].reshape(1, -1),
                            lp['ln_b'].reshape(1, -1), awh_next)

    t, g = _final_pass(h, params['tW1'], params['tb1'].reshape(1, -1),
                       params['tW2'], params['tb2'].reshape(1, -1),
                       params['gW1'], params['gb1'].reshape(1, -1),
                       params['gW2'], params['gb2'].reshape(1, -1))
    return t.reshape(-1), h, g.reshape(-1)


# speedup vs baseline: 1.6998x; 1.0640x over previous
import functools

import jax
import jax.numpy as jnp
from jax import lax
from jax.experimental import pallas as pl
from jax.experimental.pallas import tpu as pltpu
from jax.experimental.pallas import tpu_sc as plsc

BF = jnp.bfloat16
F32 = jnp.float32
EB = 1280
NB = 10000


def _lrelu(v):
    return jnp.where(v >= 0, v, 0.2 * v)


def _sc_gather(table, idx, chunk=80):
    n, d = table.shape
    e = idx.shape[0]
    info = plsc.get_sparse_core_info()
    nw = info.num_cores * info.num_subcores
    per_w = e // nw
    n_chunks = per_w // chunk
    mesh = plsc.VectorSubcoreMesh(core_axis_name="c", subcore_axis_name="s")

    @functools.partial(
        pl.kernel,
        out_type=jax.ShapeDtypeStruct((e, d), table.dtype),
        mesh=mesh,
        scratch_types=[
            pltpu.VMEM((chunk,), jnp.int32),
            pltpu.VMEM((chunk, d), table.dtype),
            pltpu.SemaphoreType.DMA,
        ],
    )
    def k(table_hbm, idx_hbm, out_hbm, idx_v, rows_v, sem):
        wid = lax.axis_index("s") * info.num_cores + lax.axis_index("c")
        base = wid * per_w

        def body(j, carry):
            b = base + j * chunk
            pltpu.sync_copy(idx_hbm.at[pl.ds(b, chunk)], idx_v)
            pltpu.async_copy(table_hbm.at[idx_v], rows_v, sem).wait()
            pltpu.sync_copy(rows_v, out_hbm.at[pl.ds(b, chunk), :])
            return carry

        lax.fori_loop(0, n_chunks, body, 0)

    return k(table, idx)


def _sc_gather_small(table, idx, chunk=80):
    n, d = table.shape
    e = idx.shape[0]
    info = plsc.get_sparse_core_info()
    nw = info.num_cores * info.num_subcores
    per_w = e // nw
    n_chunks = per_w // chunk
    mesh = plsc.VectorSubcoreMesh(core_axis_name="c", subcore_axis_name="s")

    @functools.partial(
        pl.kernel,
        out_type=jax.ShapeDtypeStruct((e * d,), table.dtype),
        mesh=mesh,
        scratch_types=[
            pltpu.VMEM((n * d,), table.dtype),
            pltpu.VMEM((chunk,), jnp.int32),
            pltpu.VMEM((chunk * d,), table.dtype),
        ],
        compiler_params=pltpu.CompilerParams(needs_layout_passes=False),
    )
    def k(table_hbm, idx_hbm, out_hbm, tab_v, idx_v, obuf_v):
        wid = lax.axis_index("s") * info.num_cores + lax.axis_index("c")
        base = wid * per_w
        pltpu.sync_copy(table_hbm, tab_v)

        def body(j, carry):
            b = base + j * chunk
            pltpu.sync_copy(idx_hbm.at[pl.ds(b, chunk)], idx_v)
            for g in range(chunk // 16):
                t = idx_v[pl.ds(g * 16, 16)] * d
                eloc = (lax.iota(jnp.int32, 16) + (g * 16)) * d
                for c in range(4):
                    vals = plsc.load_gather(tab_v, [t + c])
                    plsc.store_scatter(obuf_v, [eloc + c], vals)
            pltpu.sync_copy(obuf_v, out_hbm.at[pl.ds(b * d, chunk * d)])
            return carry

        lax.fori_loop(0, n_chunks, body, 0)

    return k(table.reshape(-1), idx).reshape(e, d)


def _sc_scatter_add(w_t, tgt, n_nodes, chunk=800):
    d, e = w_t.shape
    info = plsc.get_sparse_core_info()
    nc, ns = info.num_cores, info.num_subcores
    cs = d // (nc * ns)
    n_chunks = e // chunk
    zeros = jnp.zeros((cs * n_nodes,), w_t.dtype)
    mesh = plsc.VectorSubcoreMesh(core_axis_name="c", subcore_axis_name="s")

    @functools.partial(
        pl.kernel,
        out_type=jax.ShapeDtypeStruct((d * n_nodes,), w_t.dtype),
        mesh=mesh,
        scratch_types=[
            pltpu.VMEM((2 * chunk,), jnp.int32),
            pltpu.VMEM((2 * cs * chunk,), w_t.dtype),
            pltpu.VMEM((cs * n_nodes,), w_t.dtype),
            pltpu.SemaphoreType.DMA,
            pltpu.SemaphoreType.DMA,
        ],
        compiler_params=pltpu.CompilerParams(needs_layout_passes=False),
    )
    def k(w_hbm, tgt_hbm, zs_hbm, out_hbm, idx_v, rows_v, acc, sem0, sem1):
        cid = lax.axis_index("c")
        sid = lax.axis_index("s")
        col0 = (cid * ns + sid) * cs
        pltpu.sync_copy(zs_hbm, acc)
        sems = [sem0, sem1]

        def fire(j, b):
            eb = j * chunk
            pltpu.async_copy(tgt_hbm.at[pl.ds(eb, chunk)],
                             idx_v.at[pl.ds(b * chunk, chunk)], sems[b])
            for c in range(cs):
                pltpu.async_copy(
                    w_hbm.at[pl.ds((col0 + c) * e + eb, chunk)],
                    rows_v.at[pl.ds((b * cs + c) * chunk, chunk)], sems[b])

        def drain(j, b):
            eb = j * chunk
            pltpu.make_async_copy(
                tgt_hbm.at[pl.ds(eb, chunk)],
                idx_v.at[pl.ds(b * chunk, chunk)], sems[b]).wait()
            for c in range(cs):
                pltpu.make_async_copy(
                    w_hbm.at[pl.ds((col0 + c) * e + eb, chunk)],
                    rows_v.at[pl.ds((b * cs + c) * chunk, chunk)],
                    sems[b]).wait()

        def process(b):
            for g in range(chunk // 16):
                ids = idx_v[pl.ds(b * chunk + g * 16, 16)]
                for c in range(cs):
                    vals = rows_v[pl.ds((b * cs + c) * chunk + g * 16, 16)]
                    plsc.addupdate_scatter(acc, [ids], vals)
                    ids = ids + n_nodes

        fire(0, 0)

        def body(i, carry):
            ja = 2 * i

            @pl.when(ja + 1 < n_chunks)
            def _():
                fire(ja + 1, 1)

            drain(ja, 0)
            process(0)

            @pl.when(ja + 2 < n_chunks)
            def _():
                fire(ja + 2, 0)

            @pl.when(ja + 1 < n_chunks)
            def _():
                drain(ja + 1, 1)
                process(1)

            return carry

        lax.fori_loop(0, (n_chunks + 1) // 2, body, 0)
        for c in range(cs):
            pltpu.sync_copy(
                acc.at[pl.ds(c * n_nodes, n_nodes)],
                out_hbm.at[pl.ds((col0 + c) * n_nodes, n_nodes)])

    return k(w_t.reshape(-1), tgt, zeros).reshape(d, n_nodes)


def _dot(a, b):
    return jax.lax.dot(a.astype(BF), b.astype(BF), preferred_element_type=F32)


def _dot_t(a_t, b):
    return jax.lax.dot_general(
        a_t.astype(BF), b.astype(BF),
        dimension_numbers=(((0,), (0,)), ((), ())),
        preferred_element_type=F32)


def _row(i):
    return (i, 0)


def _row3(i):
    return (i, 0, 0)


def _fix(i):
    return (0, 0)


def _specs(block_rows, row_shapes, fixed_shapes):
    sp = [pl.BlockSpec((block_rows, s), _row) for s in row_shapes]
    sp += [pl.BlockSpec(s, _fix) for s in fixed_shapes]
    return sp


def _pass_a_body(nb, ea, tht, w1a, w1b, b1, w2, awm, bmax_ref, bsum_ref,
                 lg_ref, lbf_ref, c2_ref):
    i = pl.program_id(0)

    @pl.when(i == 0)
    def _():
        c2_ref[...] = _dot(w2[...], awm[...])

    pre = _dot(nb[...], w1a[...]) + _dot(ea[...], w1b[...]) + b1[...]
    L = _lrelu(pre)
    lbf_ref[...] = L.astype(BF)
    lg = _dot(L, c2_ref[...]) + tht[...][:, :4]
    lg_ref[...] = lg
    mx = jnp.max(lg, axis=0, keepdims=True)
    sm = jnp.sum(jnp.exp(lg - mx), axis=0, keepdims=True)
    bmax_ref[...] = mx[None]
    bsum_ref[...] = sm[None]


def _pass_a(nb, ea, tht, w1a, w1b, b1, w2, awm, off=0):
    e = nb.shape[0]
    g = e // EB
    hh = w2.shape[1]
    row_o = lambda i: (i + off, 0)
    return pl.pallas_call(
        _pass_a_body,
        grid=(g,),
        in_specs=([pl.BlockSpec((EB, nb.shape[1]), _row),
                   pl.BlockSpec((EB, ea.shape[1]), row_o),
                   pl.BlockSpec((EB, tht.shape[1]), _row)] +
                  [pl.BlockSpec(x.shape, _fix)
                   for x in (w1a, w1b, b1, w2, awm)]),
        out_specs=[pl.BlockSpec((1, 1, 4), _row3),
                   pl.BlockSpec((1, 1, 4), _row3),
                   pl.BlockSpec((EB, 4), _row),
                   pl.BlockSpec((EB, hh), _row)],
        out_shape=[jax.ShapeDtypeStruct((g, 1, 4), F32),
                   jax.ShapeDtypeStruct((g, 1, 4), F32),
                   jax.ShapeDtypeStruct((e, 4), F32),
                   jax.ShapeDtypeStruct((e, hh), BF)],
        scratch_shapes=[pltpu.VMEM((w2.shape[1], 4), F32)],
    )(nb, ea, tht, w1a, w1b, b1, w2, awm)


def _pass_c_body(lbf, lg, bmax, bsum, w2, b2, w_ref):
    m = jax.lax.dot(lbf[...], w2[...].astype(BF),
                    preferred_element_type=F32) + b2[...]
    bm = bmax[...][:, 0, :]
    bs = bsum[...][:, 0, :]
    gmax = jnp.max(bm, axis=0, keepdims=True)
    z = jnp.sum(bs * jnp.exp(bm - gmax), axis=0, keepdims=True)
    a = jnp.exp(lg[...] - gmax) / z
    h = w2.shape[1] // 4
    acc = a[:, 0:1] * m[:, :h]
    for hh in range(1, 4):
        acc = acc + a[:, hh:hh + 1] * m[:, hh * h:(hh + 1) * h]
    w_ref[...] = jnp.transpose(0.25 * acc)


def _pass_c(lbf, lg, bmax, bsum, w2, b2, off, e_half):
    g = e_half // EB
    h = w2.shape[1] // 4
    row_o = lambda i: (i + off, 0)
    return pl.pallas_call(
        _pass_c_body,
        grid=(g,),
        in_specs=([pl.BlockSpec((EB, lbf.shape[1]), row_o),
                   pl.BlockSpec((EB, 4), row_o),
                   pl.BlockSpec(bmax.shape, lambda i: (0, 0, 0)),
                   pl.BlockSpec(bsum.shape, lambda i: (0, 0, 0))] +
                  [pl.BlockSpec(x.shape, _fix) for x in (w2, b2)]),
        out_specs=pl.BlockSpec((h, EB), lambda i: (0, i)),
        out_shape=jax.ShapeDtypeStruct((h, e_half), F32),
    )(lbf, lg, bmax, bsum, w2, b2)


def _node_body(hb, agg1_t, agg2_t, u1, u2, ub, lng, lnb, awh, hn_ref,
               hbf_ref, thp_ref):
    u = (_dot(hb[...], u1[...]) +
         _dot_t(agg1_t[...] + agg2_t[...], u2[...]) + ub[...])
    mu = jnp.mean(u, axis=-1, keepdims=True)
    d = u - mu
    var = jnp.mean(d * d, axis=-1, keepdims=True)
    un = d / jnp.sqrt(var + 1e-5) * lng[...] + lnb[...]
    hn = hb[...] + _lrelu(un)
    hn_ref[...] = hn
    hbf_ref[...] = hn.astype(BF)
    th = _dot(hn, awh[...])
    thp_ref[...] = jnp.concatenate(
        [th, jnp.zeros((th.shape[0], 4), F32)], axis=-1)


def _node_pass(h, agg1_t, agg2_t, u1, u2, ub, lng, lnb, awh):
    n, hd = h.shape
    g = n // NB
    return pl.pallas_call(
        _node_body,
        grid=(g,),
        in_specs=([pl.BlockSpec((NB, hd), _row),
                   pl.BlockSpec((hd, NB), lambda i: (0, i)),
                   pl.BlockSpec((hd, NB), lambda i: (0, i))] +
                  [pl.BlockSpec(x.shape, _fix)
                   for x in (u1, u2, ub, lng, lnb, awh)]),
        out_specs=[pl.BlockSpec((NB, hd), _row), pl.BlockSpec((NB, hd), _row),
                   pl.BlockSpec((NB, 8), _row)],
        out_shape=[jax.ShapeDtypeStruct((n, hd), F32),
                   jax.ShapeDtypeStruct((n, hd), BF),
                   jax.ShapeDtypeStruct((n, 8), F32)],
        compiler_params=pltpu.CompilerParams(
            vmem_limit_bytes=100 * 1024 * 1024),
    )(h, agg1_t, agg2_t, u1, u2, ub, lng, lnb, awh)


def _init_body(xb, inw, inb, awh, h_ref, hbf_ref, thp_ref):
    h = _dot(xb[...], inw[...]) + inb[...]
    h_ref[...] = h
    hbf_ref[...] = h.astype(BF)
    th = _dot(h, awh[...])
    thp_ref[...] = jnp.concatenate(
        [th, jnp.zeros((th.shape[0], 4), F32)], axis=-1)


def _init_pass(x, inw, inb, awh):
    n, din = x.shape
    hd = inw.shape[1]
    g = n // NB
    return pl.pallas_call(
        _init_body,
        grid=(g,),
        in_specs=_specs(NB, [din], [inw.shape, inb.shape, awh.shape]),
        out_specs=[pl.BlockSpec((NB, hd), _row), pl.BlockSpec((NB, hd), _row),
                   pl.BlockSpec((NB, 8), _row)],
        out_shape=[jax.ShapeDtypeStruct((n, hd), F32),
                   jax.ShapeDtypeStruct((n, hd), BF),
                   jax.ShapeDtypeStruct((n, 8), F32)],
    )(x, inw, inb, awh)


def _final_body(hb, tw1, tb1, tw2, tb2, gw1, gb1, gw2, gb2, t_ref, g_ref,
                acc_ref):
    i = pl.program_id(0)
    n_total = pl.num_programs(0) * hb.shape[0]
    t1 = jnp.maximum(_dot(hb[...], tw1[...]) + tb1[...], 0.0)
    t_ref[...] = _dot(t1, tw2[...]) + tb2[...]

    @pl.when(i == 0)
    def _():
        acc_ref[...] = jnp.zeros_like(acc_ref)

    acc_ref[...] += jnp.sum(hb[...], axis=0, keepdims=True)

    @pl.when(i == pl.num_programs(0) - 1)
    def _():
        ge = acc_ref[...] / n_total
        g1 = jnp.maximum(_dot(ge, gw1[...]) + gb1[...], 0.0)
        g_ref[...] = _dot(g1, gw2[...]) + gb2[...]


def _final_pass(h, tw1, tb1, tw2, tb2, gw1, gb1, gw2, gb2):
    n, hd = h.shape
    g = n // NB
    return pl.pallas_call(
        _final_body,
        grid=(g,),
        in_specs=_specs(NB, [hd],
                        [tw1.shape, tb1.shape, tw2.shape, tb2.shape,
                         gw1.shape, gb1.shape, gw2.shape, gb2.shape]),
        out_specs=[pl.BlockSpec((NB, 1), _row),
                   pl.BlockSpec((1, 4), lambda i: (0, 0))],
        out_shape=[jax.ShapeDtypeStruct((n, 1), F32),
                   jax.ShapeDtypeStruct((1, 4), F32)],
        scratch_shapes=[pltpu.VMEM((1, hd), F32)],
    )(h, tw1, tb1, tw2, tb2, gw1, gb1, gw2, gb2)


def kernel(x, edge_index, edge_attr, params):
    src, tgt = edge_index[0], edge_index[1]
    hd = params['inW'].shape[1]
    n = x.shape[0]
    layers = params['layers']

    h, hbf, thp = _init_pass(x, params['inW'], params['inb'].reshape(1, -1),
                             layers[0]['aW'][layers[0]['mW2'].shape[1]:])
    for li, lp in enumerate(layers):
        hh = lp['mW2'].shape[1]
        w1a = lp['mW1'][:hd]
        w1b = lp['mW1'][hd:]
        b1 = lp['mb1'].reshape(1, -1)
        b2 = lp['mb2'].reshape(1, -1)
        awm = lp['aW'][:hh]
        ab = lp['ab'].reshape(1, -1)

        e2 = edge_attr.shape[0] // 2
        g2 = e2 // EB
        nb1 = _sc_gather(h, src[:e2], chunk=200)
        tht1 = _sc_gather_small(thp, tgt[:e2], chunk=200)
        nb2 = _sc_gather(h, src[e2:], chunk=200)
        tht2 = _sc_gather_small(thp, tgt[e2:], chunk=200)
        bmax1, bsum1, lg1, lbf1 = _pass_a(nb1, edge_attr, tht1, w1a, w1b, b1,
                                          lp['mW2'], awm, 0)
        bmax2, bsum2, lg2, lbf2 = _pass_a(nb2, edge_attr, tht2, w1a, w1b, b1,
                                          lp['mW2'], awm, g2)
        bmax = jnp.concatenate([bmax1, bmax2], axis=0)
        bsum = jnp.concatenate([bsum1, bsum2], axis=0)
        w1t = _pass_c(lbf1, lg1, bmax, bsum, lp['mW2'], b2, 0, e2)
        agg1 = _sc_scatter_add(w1t, tgt[:e2], n)
        w2t = _pass_c(lbf2, lg2, bmax, bsum, lp['mW2'], b2, 0, e2)
        agg2 = _sc_scatter_add(w2t, tgt[e2:], n)
        nxt = layers[li + 1] if li + 1 < len(layers) else None
        awh_next = (nxt['aW'][nxt['mW2'].shape[1]:] if nxt is not None
                    else jnp.zeros((hd, 4), F32))
        h, hbf, thp = _node_pass(h, agg1, agg2, lp['uW'][:hd], lp['uW'][hd:],
                            lp['ub'].reshape(1, -1),
                            lp['ln_g'].reshape(1, -1),
                            lp['ln_b'].reshape(1, -1), awh_next)

    t, g = _final_pass(h, params['tW1'], params['tb1'].reshape(1, -1),
                       params['tW2'], params['tb2'].reshape(1, -1),
                       params['gW1'], params['gb1'].reshape(1, -1),
                       params['gW2'], params['gb2'].reshape(1, -1))
    return t.reshape(-1), h, g.reshape(-1)


# speedup vs baseline: 1.8233x; 1.0727x over previous
import functools

import jax
import jax.numpy as jnp
from jax import lax
from jax.experimental import pallas as pl
from jax.experimental.pallas import tpu as pltpu
from jax.experimental.pallas import tpu_sc as plsc

BF = jnp.bfloat16
F32 = jnp.float32
EB = 3200
NB = 10000


def _lrelu(v):
    return jnp.where(v >= 0, v, 0.2 * v)


def _sc_gather(table, idx, chunk=80):
    n, d = table.shape
    e = idx.shape[0]
    info = plsc.get_sparse_core_info()
    nw = info.num_cores * info.num_subcores
    per_w = e // nw
    n_chunks = per_w // chunk
    mesh = plsc.VectorSubcoreMesh(core_axis_name="c", subcore_axis_name="s")

    @functools.partial(
        pl.kernel,
        out_type=jax.ShapeDtypeStruct((e, d), table.dtype),
        mesh=mesh,
        scratch_types=[
            pltpu.VMEM((chunk,), jnp.int32),
            pltpu.VMEM((chunk, d), table.dtype),
            pltpu.SemaphoreType.DMA,
        ],
    )
    def k(table_hbm, idx_hbm, out_hbm, idx_v, rows_v, sem):
        wid = lax.axis_index("s") * info.num_cores + lax.axis_index("c")
        base = wid * per_w

        def body(j, carry):
            b = base + j * chunk
            pltpu.sync_copy(idx_hbm.at[pl.ds(b, chunk)], idx_v)
            pltpu.async_copy(table_hbm.at[idx_v], rows_v, sem).wait()
            pltpu.sync_copy(rows_v, out_hbm.at[pl.ds(b, chunk), :])
            return carry

        lax.fori_loop(0, n_chunks, body, 0)

    return k(table, idx)


def _sc_gather_small(table, idx, chunk=80):
    n, d = table.shape
    e = idx.shape[0]
    info = plsc.get_sparse_core_info()
    nw = info.num_cores * info.num_subcores
    per_w = e // nw
    n_chunks = per_w // chunk
    mesh = plsc.VectorSubcoreMesh(core_axis_name="c", subcore_axis_name="s")

    @functools.partial(
        pl.kernel,
        out_type=jax.ShapeDtypeStruct((e * d,), table.dtype),
        mesh=mesh,
        scratch_types=[
            pltpu.VMEM((n * d,), table.dtype),
            pltpu.VMEM((chunk,), jnp.int32),
            pltpu.VMEM((chunk * d,), table.dtype),
        ],
        compiler_params=pltpu.CompilerParams(needs_layout_passes=False),
    )
    def k(table_hbm, idx_hbm, out_hbm, tab_v, idx_v, obuf_v):
        wid = lax.axis_index("s") * info.num_cores + lax.axis_index("c")
        base = wid * per_w
        pltpu.sync_copy(table_hbm, tab_v)

        def body(j, carry):
            b = base + j * chunk
            pltpu.sync_copy(idx_hbm.at[pl.ds(b, chunk)], idx_v)
            for g in range(chunk // 16):
                t = idx_v[pl.ds(g * 16, 16)] * d
                eloc = (lax.iota(jnp.int32, 16) + (g * 16)) * d
                for c in range(4):
                    vals = plsc.load_gather(tab_v, [t + c])
                    plsc.store_scatter(obuf_v, [eloc + c], vals)
            pltpu.sync_copy(obuf_v, out_hbm.at[pl.ds(b * d, chunk * d)])
            return carry

        lax.fori_loop(0, n_chunks, body, 0)

    return k(table.reshape(-1), idx).reshape(e, d)


def _sc_scatter_add(w_t, tgt, n_nodes, chunk=1600):
    d, e = w_t.shape
    info = plsc.get_sparse_core_info()
    nc, ns = info.num_cores, info.num_subcores
    cs = d // (nc * ns)
    n_chunks = e // chunk
    zeros = jnp.zeros((cs * n_nodes,), w_t.dtype)
    mesh = plsc.VectorSubcoreMesh(core_axis_name="c", subcore_axis_name="s")

    @functools.partial(
        pl.kernel,
        out_type=jax.ShapeDtypeStruct((d * n_nodes,), w_t.dtype),
        mesh=mesh,
        scratch_types=[
            pltpu.VMEM((2 * chunk,), jnp.int32),
            pltpu.VMEM((2 * cs * chunk,), w_t.dtype),
            pltpu.VMEM((cs * n_nodes,), w_t.dtype),
            pltpu.SemaphoreType.DMA,
            pltpu.SemaphoreType.DMA,
        ],
        compiler_params=pltpu.CompilerParams(needs_layout_passes=False),
    )
    def k(w_hbm, tgt_hbm, zs_hbm, out_hbm, idx_v, rows_v, acc, sem0, sem1):
        cid = lax.axis_index("c")
        sid = lax.axis_index("s")
        col0 = (cid * ns + sid) * cs
        pltpu.sync_copy(zs_hbm, acc)
        sems = [sem0, sem1]

        def fire(j, b):
            eb = j * chunk
            pltpu.async_copy(tgt_hbm.at[pl.ds(eb, chunk)],
                             idx_v.at[pl.ds(b * chunk, chunk)], sems[b])
            for c in range(cs):
                pltpu.async_copy(
                    w_hbm.at[pl.ds((col0 + c) * e + eb, chunk)],
                    rows_v.at[pl.ds((b * cs + c) * chunk, chunk)], sems[b])

        def drain(j, b):
            eb = j * chunk
            pltpu.make_async_copy(
                tgt_hbm.at[pl.ds(eb, chunk)],
                idx_v.at[pl.ds(b * chunk, chunk)], sems[b]).wait()
            for c in range(cs):
                pltpu.make_async_copy(
                    w_hbm.at[pl.ds((col0 + c) * e + eb, chunk)],
                    rows_v.at[pl.ds((b * cs + c) * chunk, chunk)],
                    sems[b]).wait()

        def process(b):
            for g in range(chunk // 16):
                ids = idx_v[pl.ds(b * chunk + g * 16, 16)]
                for c in range(cs):
                    vals = rows_v[pl.ds((b * cs + c) * chunk + g * 16, 16)]
                    plsc.addupdate_scatter(acc, [ids], vals)
                    ids = ids + n_nodes

        fire(0, 0)

        def body(i, carry):
            ja = 2 * i

            @pl.when(ja + 1 < n_chunks)
            def _():
                fire(ja + 1, 1)

            drain(ja, 0)
            process(0)

            @pl.when(ja + 2 < n_chunks)
            def _():
                fire(ja + 2, 0)

            @pl.when(ja + 1 < n_chunks)
            def _():
                drain(ja + 1, 1)
                process(1)

            return carry

        lax.fori_loop(0, (n_chunks + 1) // 2, body, 0)
        for c in range(cs):
            pltpu.sync_copy(
                acc.at[pl.ds(c * n_nodes, n_nodes)],
                out_hbm.at[pl.ds((col0 + c) * n_nodes, n_nodes)])

    return k(w_t.reshape(-1), tgt, zeros).reshape(d, n_nodes)


def _dot(a, b):
    return jax.lax.dot(a.astype(BF), b.astype(BF), preferred_element_type=F32)


def _dot_t(a_t, b):
    return jax.lax.dot_general(
        a_t.astype(BF), b.astype(BF),
        dimension_numbers=(((0,), (0,)), ((), ())),
        preferred_element_type=F32)


def _row(i):
    return (i, 0)


def _row3(i):
    return (i, 0, 0)


def _fix(i):
    return (0, 0)


def _specs(block_rows, row_shapes, fixed_shapes):
    sp = [pl.BlockSpec((block_rows, s), _row) for s in row_shapes]
    sp += [pl.BlockSpec(s, _fix) for s in fixed_shapes]
    return sp


def _pass_a_body(nb, ea, tht, w1a, w1b, b1, w2, awm, bmax_ref, bsum_ref,
                 lg_ref, lbf_ref, c2_ref):
    i = pl.program_id(0)

    @pl.when(i == 0)
    def _():
        c2_ref[...] = _dot(w2[...], awm[...])

    pre = _dot(nb[...], w1a[...]) + _dot(ea[...], w1b[...]) + b1[...]
    L = _lrelu(pre)
    lbf_ref[...] = L.astype(BF)
    lg = _dot(L, c2_ref[...]) + tht[...][:, :4]
    lg_ref[...] = lg
    mx = jnp.max(lg, axis=0, keepdims=True)
    sm = jnp.sum(jnp.exp(lg - mx), axis=0, keepdims=True)
    bmax_ref[...] = mx[None]
    bsum_ref[...] = sm[None]


def _pass_a(nb, ea, tht, w1a, w1b, b1, w2, awm, off=0):
    e = nb.shape[0]
    g = e // EB
    hh = w2.shape[1]
    row_o = lambda i: (i + off, 0)
    return pl.pallas_call(
        _pass_a_body,
        grid=(g,),
        in_specs=([pl.BlockSpec((EB, nb.shape[1]), _row),
                   pl.BlockSpec((EB, ea.shape[1]), row_o),
                   pl.BlockSpec((EB, tht.shape[1]), _row)] +
                  [pl.BlockSpec(x.shape, _fix)
                   for x in (w1a, w1b, b1, w2, awm)]),
        out_specs=[pl.BlockSpec((1, 1, 4), _row3),
                   pl.BlockSpec((1, 1, 4), _row3),
                   pl.BlockSpec((EB, 4), _row),
                   pl.BlockSpec((EB, hh), _row)],
        out_shape=[jax.ShapeDtypeStruct((g, 1, 4), F32),
                   jax.ShapeDtypeStruct((g, 1, 4), F32),
                   jax.ShapeDtypeStruct((e, 4), F32),
                   jax.ShapeDtypeStruct((e, hh), BF)],
        scratch_shapes=[pltpu.VMEM((w2.shape[1], 4), F32)],
        compiler_params=pltpu.CompilerParams(
            vmem_limit_bytes=100 * 1024 * 1024),
    )(nb, ea, tht, w1a, w1b, b1, w2, awm)


def _pass_c_body(lbf, lg, bmax, bsum, w2, b2, w_ref):
    m = jax.lax.dot(lbf[...], w2[...].astype(BF),
                    preferred_element_type=F32) + b2[...]
    bm = bmax[...][:, 0, :]
    bs = bsum[...][:, 0, :]
    gmax = jnp.max(bm, axis=0, keepdims=True)
    z = jnp.sum(bs * jnp.exp(bm - gmax), axis=0, keepdims=True)
    a = jnp.exp(lg[...] - gmax) / z
    h = w2.shape[1] // 4
    acc = a[:, 0:1] * m[:, :h]
    for hh in range(1, 4):
        acc = acc + a[:, hh:hh + 1] * m[:, hh * h:(hh + 1) * h]
    w_ref[...] = jnp.transpose(0.25 * acc)


def _pass_c(lbf, lg, bmax, bsum, w2, b2, off, e_half):
    g = e_half // EB
    h = w2.shape[1] // 4
    row_o = lambda i: (i + off, 0)
    return pl.pallas_call(
        _pass_c_body,
        grid=(g,),
        in_specs=([pl.BlockSpec((EB, lbf.shape[1]), row_o),
                   pl.BlockSpec((EB, 4), row_o),
                   pl.BlockSpec(bmax.shape, lambda i: (0, 0, 0)),
                   pl.BlockSpec(bsum.shape, lambda i: (0, 0, 0))] +
                  [pl.BlockSpec(x.shape, _fix) for x in (w2, b2)]),
        out_specs=pl.BlockSpec((h, EB), lambda i: (0, i)),
        out_shape=jax.ShapeDtypeStruct((h, e_half), F32),
        compiler_params=pltpu.CompilerParams(
            vmem_limit_bytes=100 * 1024 * 1024),
    )(lbf, lg, bmax, bsum, w2, b2)


def _node_body(hb, agg1_t, agg2_t, u1, u2, ub, lng, lnb, awh, hn_ref,
               hbf_ref, thp_ref):
    u = (_dot(hb[...], u1[...]) +
         _dot_t(agg1_t[...] + agg2_t[...], u2[...]) + ub[...])
    mu = jnp.mean(u, axis=-1, keepdims=True)
    d = u - mu
    var = jnp.mean(d * d, axis=-1, keepdims=True)
    un = d / jnp.sqrt(var + 1e-5) * lng[...] + lnb[...]
    hn = hb[...] + _lrelu(un)
    hn_ref[...] = hn
    hbf_ref[...] = hn.astype(BF)
    th = _dot(hn, awh[...])
    thp_ref[...] = jnp.concatenate(
        [th, jnp.zeros((th.shape[0], 4), F32)], axis=-1)


def _node_pass(h, agg1_t, agg2_t, u1, u2, ub, lng, lnb, awh):
    n, hd = h.shape
    g = n // NB
    return pl.pallas_call(
        _node_body,
        grid=(g,),
        in_specs=([pl.BlockSpec((NB, hd), _row),
                   pl.BlockSpec((hd, NB), lambda i: (0, i)),
                   pl.BlockSpec((hd, NB), lambda i: (0, i))] +
                  [pl.BlockSpec(x.shape, _fix)
                   for x in (u1, u2, ub, lng, lnb, awh)]),
        out_specs=[pl.BlockSpec((NB, hd), _row), pl.BlockSpec((NB, hd), _row),
                   pl.BlockSpec((NB, 8), _row)],
        out_shape=[jax.ShapeDtypeStruct((n, hd), F32),
                   jax.ShapeDtypeStruct((n, hd), BF),
                   jax.ShapeDtypeStruct((n, 8), F32)],
        compiler_params=pltpu.CompilerParams(
            vmem_limit_bytes=100 * 1024 * 1024),
    )(h, agg1_t, agg2_t, u1, u2, ub, lng, lnb, awh)


def _init_body(xb, inw, inb, awh, h_ref, hbf_ref, thp_ref):
    h = _dot(xb[...], inw[...]) + inb[...]
    h_ref[...] = h
    hbf_ref[...] = h.astype(BF)
    th = _dot(h, awh[...])
    thp_ref[...] = jnp.concatenate(
        [th, jnp.zeros((th.shape[0], 4), F32)], axis=-1)


def _init_pass(x, inw, inb, awh):
    n, din = x.shape
    hd = inw.shape[1]
    g = n // NB
    return pl.pallas_call(
        _init_body,
        grid=(g,),
        in_specs=_specs(NB, [din], [inw.shape, inb.shape, awh.shape]),
        out_specs=[pl.BlockSpec((NB, hd), _row), pl.BlockSpec((NB, hd), _row),
                   pl.BlockSpec((NB, 8), _row)],
        out_shape=[jax.ShapeDtypeStruct((n, hd), F32),
                   jax.ShapeDtypeStruct((n, hd), BF),
                   jax.ShapeDtypeStruct((n, 8), F32)],
    )(x, inw, inb, awh)


def _final_body(hb, tw1, tb1, tw2, tb2, gw1, gb1, gw2, gb2, t_ref, g_ref,
                acc_ref):
    i = pl.program_id(0)
    n_total = pl.num_programs(0) * hb.shape[0]
    t1 = jnp.maximum(_dot(hb[...], tw1[...]) + tb1[...], 0.0)
    t_ref[...] = _dot(t1, tw2[...]) + tb2[...]

    @pl.when(i == 0)
    def _():
        acc_ref[...] = jnp.zeros_like(acc_ref)

    acc_ref[...] += jnp.sum(hb[...], axis=0, keepdims=True)

    @pl.when(i == pl.num_programs(0) - 1)
    def _():
        ge = acc_ref[...] / n_total
        g1 = jnp.maximum(_dot(ge, gw1[...]) + gb1[...], 0.0)
        g_ref[...] = _dot(g1, gw2[...]) + gb2[...]


def _final_pass(h, tw1, tb1, tw2, tb2, gw1, gb1, gw2, gb2):
    n, hd = h.shape
    g = n // NB
    return pl.pallas_call(
        _final_body,
        grid=(g,),
        in_specs=_specs(NB, [hd],
                        [tw1.shape, tb1.shape, tw2.shape, tb2.shape,
                         gw1.shape, gb1.shape, gw2.shape, gb2.shape]),
        out_specs=[pl.BlockSpec((NB, 1), _row),
                   pl.BlockSpec((1, 4), lambda i: (0, 0))],
        out_shape=[jax.ShapeDtypeStruct((n, 1), F32),
                   jax.ShapeDtypeStruct((1, 4), F32)],
        scratch_shapes=[pltpu.VMEM((1, hd), F32)],
    )(h, tw1, tb1, tw2, tb2, gw1, gb1, gw2, gb2)


def kernel(x, edge_index, edge_attr, params):
    src, tgt = edge_index[0], edge_index[1]
    hd = params['inW'].shape[1]
    n = x.shape[0]
    layers = params['layers']

    h, hbf, thp = _init_pass(x, params['inW'], params['inb'].reshape(1, -1),
                             layers[0]['aW'][layers[0]['mW2'].shape[1]:])
    for li, lp in enumerate(layers):
        hh = lp['mW2'].shape[1]
        w1a = lp['mW1'][:hd]
        w1b = lp['mW1'][hd:]
        b1 = lp['mb1'].reshape(1, -1)
        b2 = lp['mb2'].reshape(1, -1)
        awm = lp['aW'][:hh]
        ab = lp['ab'].reshape(1, -1)

        e2 = edge_attr.shape[0] // 2
        g2 = e2 // EB
        nb1 = _sc_gather(h, src[:e2], chunk=200)
        tht1 = _sc_gather_small(thp, tgt[:e2], chunk=200)
        nb2 = _sc_gather(h, src[e2:], chunk=200)
        tht2 = _sc_gather_small(thp, tgt[e2:], chunk=200)
        bmax1, bsum1, lg1, lbf1 = _pass_a(nb1, edge_attr, tht1, w1a, w1b, b1,
                                          lp['mW2'], awm, 0)
        bmax2, bsum2, lg2, lbf2 = _pass_a(nb2, edge_attr, tht2, w1a, w1b, b1,
                                          lp['mW2'], awm, g2)
        bmax = jnp.concatenate([bmax1, bmax2], axis=0)
        bsum = jnp.concatenate([bsum1, bsum2], axis=0)
        w1t = _pass_c(lbf1, lg1, bmax, bsum, lp['mW2'], b2, 0, e2)
        agg1 = _sc_scatter_add(w1t, tgt[:e2], n)
        w2t = _pass_c(lbf2, lg2, bmax, bsum, lp['mW2'], b2, 0, e2)
        agg2 = _sc_scatter_add(w2t, tgt[e2:], n)
        nxt = layers[li + 1] if li + 1 < len(layers) else None
        awh_next = (nxt['aW'][nxt['mW2'].shape[1]:] if nxt is not None
                    else jnp.zeros((hd, 4), F32))
        h, hbf, thp = _node_pass(h, agg1, agg2, lp['uW'][:hd], lp['uW'][hd:],
                            lp['ub'].reshape(1, -1),
                            lp['ln_g'].reshape(1, -1),
                            lp['ln_b'].reshape(1, -1), awh_next)

    t, g = _final_pass(h, params['tW1'], params['tb1'].reshape(1, -1),
                       params['tW2'], params['tb2'].reshape(1, -1),
                       params['gW1'], params['gb1'].reshape(1, -1),
                       params['gW2'], params['gb2'].reshape(1, -1))
    return t.reshape(-1), h, g.reshape(-1)
